# Initial kernel scaffold; baseline (speedup 1.0000x reference)
import jax
import jax.numpy as jnp
from jax.experimental import pallas as pl


def kernel(x, edge_index, batch, params):
    raise NotImplementedError("write your pallas kernel here")


# speedup vs baseline: 8.0905x; 8.0905x over previous
import functools

import jax
import jax.numpy as jnp
import numpy as np
from jax import lax
from jax.experimental import pallas as pl
from jax.experimental.pallas import tpu as pltpu
from jax.experimental.pallas import tpu_sc as plsc

N_NODES = 10000
N_EDGES = 160000
HID = 256
HH = 128
NG = 512
NS = 16
NC = 2
ECH = 128
NCHT = 79
E_PAD = NS * NCHT * ECH
ACCN = 10008
DUMMY = 10000
RPS = 624
TAIL = N_NODES - NS * RPS
TOFF = NS * RPS
PCH = 104
NPJ = RPS // PCH
RB = 1000
NRB = N_NODES // RB
BN_EPS = 1e-5
ALPHA = 0.01
IN_DIM = 82
_FEATURE_LENS = [44, 7, 6, 7, 2, 2, 6, 8]

_mesh = plsc.VectorSubcoreMesh(core_axis_name="c", subcore_axis_name="s")


@functools.partial(
    pl.kernel,
    out_type=jax.ShapeDtypeStruct((NC, N_NODES, HH), jnp.float32),
    mesh=_mesh,
    scratch_types=[
        pltpu.VMEM((2, 2, ECH), jnp.int32),
        pltpu.VMEM((ECH, HH), jnp.float32),
        pltpu.VMEM_SHARED((ACCN, HH), jnp.float32),
        pltpu.SemaphoreType.DMA,
        pltpu.SemaphoreType.DMA,
    ],
)
def _sc_deg(eidx, ones, zeros, out, idx_v, ones_v, acc, semA, semB):
    c = lax.axis_index("c")
    s = lax.axis_index("s")
    pltpu.sync_copy(ones, ones_v)
    pltpu.sync_copy(zeros, acc.at[pl.ds(s * RPS, RPS)])

    @pl.when(s == 0)
    def _():
        pltpu.sync_copy(zeros.at[pl.ds(0, TAIL)], acc.at[pl.ds(TOFF, TAIL)])

    plsc.subcore_barrier()

    et = eidx.at[s]
    start = c * 40
    pltpu.async_copy(et.at[start], idx_v.at[0], semA)

    def step(t, carry):
        j = start + 2 * t
        pltpu.async_copy(et.at[j + 1], idx_v.at[1], semB)
        pltpu.make_async_copy(et.at[0], idx_v.at[0], semA).wait()
        pltpu.sync_copy(ones_v, acc.at[idx_v.at[0, 1]], add=True)
        pltpu.async_copy(et.at[jnp.minimum(j + 2, NCHT - 1)], idx_v.at[0], semA)
        pltpu.make_async_copy(et.at[0], idx_v.at[1], semB).wait()
        pltpu.sync_copy(ones_v, acc.at[idx_v.at[1, 1]], add=True)
        return carry

    lax.fori_loop(0, 20 - c, step, 0)
    pltpu.make_async_copy(et.at[0], idx_v.at[0], semA).wait()

    @pl.when(c == 1)
    def _():
        pltpu.sync_copy(ones_v, acc.at[idx_v.at[0, 1]], add=True)

    plsc.subcore_barrier()
    pltpu.sync_copy(acc.at[pl.ds(s * RPS, RPS)], out.at[c, pl.ds(s * RPS, RPS)])

    @pl.when(s == 0)
    def _():
        pltpu.sync_copy(acc.at[pl.ds(TOFF, TAIL)], out.at[c, pl.ds(TOFF, TAIL)])


@functools.partial(
    pl.kernel,
    out_type=jax.ShapeDtypeStruct((NC, N_NODES, HH), jnp.float32),
    mesh=_mesh,
    scratch_types=[
        pltpu.VMEM((2, 2, ECH), jnp.int32),
        pltpu.VMEM((ECH, HH), jnp.float32),
        pltpu.VMEM((ECH, HH), jnp.float32),
        pltpu.VMEM_SHARED((ACCN, HH), jnp.float32),
        pltpu.SemaphoreType.DMA,
        pltpu.SemaphoreType.DMA,
        pltpu.SemaphoreType.DMA,
        pltpu.SemaphoreType.DMA,
    ],
)
def _sc_agg(hwp, eidx, zeros, out, idx_v, buf0, buf1, acc, sem0, sem1, semA, semB):
    c = lax.axis_index("c")
    s = lax.axis_index("s")
    pltpu.sync_copy(zeros, acc.at[pl.ds(s * RPS, RPS)])

    @pl.when(s == 0)
    def _():
        pltpu.sync_copy(zeros.at[pl.ds(0, TAIL)], acc.at[pl.ds(TOFF, TAIL)])

    plsc.subcore_barrier()

    hw_c = hwp.at[c]
    et = eidx.at[s]
    pltpu.sync_copy(et.at[0], idx_v.at[0])
    pltpu.async_copy(et.at[1], idx_v.at[1], semB)
    pltpu.async_copy(hw_c.at[idx_v.at[0, 0]], buf0, sem0)

    def step(t, carry):
        j = 2 * t
        pltpu.make_async_copy(et.at[0], idx_v.at[1], semB).wait()
        pltpu.async_copy(hw_c.at[idx_v.at[1, 0]], buf1, sem1)
        pltpu.make_async_copy(hw_c.at[idx_v.at[0, 0]], buf0, sem0).wait()
        pltpu.sync_copy(buf0, acc.at[idx_v.at[0, 1]], add=True)
        pltpu.async_copy(et.at[j + 2], idx_v.at[0], semA)
        pltpu.make_async_copy(hw_c.at[idx_v.at[0, 0]], buf1, sem1).wait()
        pltpu.sync_copy(buf1, acc.at[idx_v.at[1, 1]], add=True)
        pltpu.async_copy(et.at[jnp.minimum(j + 3, NCHT - 1)], idx_v.at[1], semB)
        pltpu.make_async_copy(et.at[0], idx_v.at[0], semA).wait()
        pltpu.async_copy(hw_c.at[idx_v.at[0, 0]], buf0, sem0)
        return carry

    lax.fori_loop(0, (NCHT - 1) // 2, step, 0)
    pltpu.make_async_copy(hw_c.at[idx_v.at[0, 0]], buf0, sem0).wait()
    pltpu.sync_copy(buf0, acc.at[idx_v.at[0, 1]], add=True)
    pltpu.make_async_copy(et.at[0], idx_v.at[1], semB).wait()
    plsc.subcore_barrier()
    pltpu.sync_copy(acc.at[pl.ds(s * RPS, RPS)], out.at[c, pl.ds(s * RPS, RPS)])

    @pl.when(s == 0)
    def _():
        pltpu.sync_copy(acc.at[pl.ds(TOFF, TAIL)], out.at[c, pl.ds(TOFF, TAIL)])


@functools.partial(
    pl.kernel,
    out_type=jax.ShapeDtypeStruct((NC, NG, HH), jnp.float32),
    mesh=_mesh,
    scratch_types=[
        pltpu.VMEM((NPJ, PCH), jnp.int32),
        pltpu.VMEM((1, TAIL), jnp.int32),
        pltpu.VMEM((PCH, HH), jnp.float32),
        pltpu.VMEM_SHARED((NG, HH), jnp.float32),
    ],
)
def _sc_pool(h, batT, batTail, zeros, out, bat_v, batt_v, buf, acc):
    c = lax.axis_index("c")
    s = lax.axis_index("s")
    gps = NG // NS
    pltpu.sync_copy(batT.at[s], bat_v)
    pltpu.sync_copy(batTail, batt_v)
    pltpu.sync_copy(zeros.at[pl.ds(0, gps)], acc.at[pl.ds(s * gps, gps)])
    plsc.subcore_barrier()
    for j in range(NPJ):
        pltpu.sync_copy(h.at[c, pl.ds(s * RPS + j * PCH, PCH)], buf)
        pltpu.sync_copy(buf, acc.at[bat_v.at[j]], add=True)

    @pl.when(s == 0)
    def _():
        pltpu.sync_copy(h.at[c, pl.ds(TOFF, TAIL)], buf.at[pl.ds(0, TAIL)])
        pltpu.sync_copy(buf.at[pl.ds(0, TAIL)], acc.at[batt_v.at[0]], add=True)

    plsc.subcore_barrier()
    pltpu.sync_copy(acc.at[pl.ds(s * gps, gps)], out.at[c, pl.ds(s * gps, gps)])


KPAD = 128
_OFFS = [0, 44, 51, 57, 64, 66, 68, 74]


def _dinv_from(deg):
    return lax.rsqrt(deg[0, :, 0] + deg[1, :, 0] + 1.0)


def _tca1_body(x_ref, w_ref, deg_ref, out_ref):
    dinv = _dinv_from(deg_ref[...])
    xv = x_ref[...]
    cols = lax.broadcasted_iota(jnp.int32, (RB, KPAD), 1)
    oh = jnp.zeros((RB, KPAD), jnp.float32)
    for i, o in enumerate(_OFFS):
        oh = oh + (cols == (xv[:, i:i + 1] + o)).astype(jnp.float32)
    t = jnp.dot(oh, w_ref[...], preferred_element_type=jnp.float32)
    t = t * dinv[:, None]
    out_ref[0] = t[:, :HH]
    out_ref[1] = t[:, HH:]


def _tca1(x, w1p, degp):
    return pl.pallas_call(
        _tca1_body,
        grid=(NRB,),
        in_specs=[
            pl.BlockSpec((RB, 8), lambda i: (i, 0)),
            pl.BlockSpec((KPAD, HID), lambda i: (0, 0)),
            pl.BlockSpec((NC, RB, HH), lambda i: (0, i, 0)),
        ],
        out_specs=pl.BlockSpec((NC, RB, HH), lambda i: (0, i, 0)),
        out_shape=jax.ShapeDtypeStruct((NC, N_NODES, HH), jnp.float32),
    )(x, w1p, degp)


def _tca_body(h_ref, w_ref, deg_ref, out_ref):
    dinv = _dinv_from(deg_ref[...])
    h = h_ref[...]
    hh = jnp.concatenate([h[0], h[1]], axis=1)
    t = jnp.dot(hh, w_ref[...], preferred_element_type=jnp.float32)
    t = t * dinv[:, None]
    out_ref[0] = t[:, :HH]
    out_ref[1] = t[:, HH:]


def _tca(h, w, degp):
    return pl.pallas_call(
        _tca_body,
        grid=(NRB,),
        in_specs=[
            pl.BlockSpec((NC, RB, HH), lambda i: (0, i, 0)),
            pl.BlockSpec((HID, HID), lambda i: (0, 0)),
            pl.BlockSpec((NC, RB, HH), lambda i: (0, i, 0)),
        ],
        out_specs=pl.BlockSpec((NC, RB, HH), lambda i: (0, i, 0)),
        out_shape=jax.ShapeDtypeStruct((NC, N_NODES, HH), jnp.float32),
    )(h, w, degp)


def _tcb_body(agg_ref, hwp_ref, deg_ref, b_ref, g_ref, be_ref, out_ref, ssum, ssq):
    p = pl.program_id(0)
    i = pl.program_id(1)
    dinv = _dinv_from(deg_ref[...])[None, :, None]
    t = (agg_ref[...] + hwp_ref[...]) * dinv + b_ref[...][:, None, :]

    @pl.when((p == 0) & (i == 0))
    def _():
        ssum[...] = jnp.zeros_like(ssum)
        ssq[...] = jnp.zeros_like(ssq)

    @pl.when(p == 0)
    def _():
        ssum[...] += t.sum(axis=1)

    @pl.when(p == 1)
    def _():
        mu = (ssum[...] / N_NODES)[:, None, :]
        d = t - mu
        ssq[...] += (d * d).sum(axis=1)

    @pl.when(p == 2)
    def _():
        mu = (ssum[...] / N_NODES)[:, None, :]
        var = (ssq[...] / N_NODES)[:, None, :]
        z = g_ref[...][:, None, :] * (t - mu) / jnp.sqrt(var + BN_EPS) \
            + be_ref[...][:, None, :]
        out_ref[...] = jnp.where(z > 0, z, ALPHA * z)


def _tcb(agg, hwp, degp, b, g, be):
    return pl.pallas_call(
        _tcb_body,
        grid=(3, NRB),
        in_specs=[
            pl.BlockSpec((NC, RB, HH), lambda p, i: (0, i, 0)),
            pl.BlockSpec((NC, RB, HH), lambda p, i: (0, i, 0)),
            pl.BlockSpec((NC, RB, HH), lambda p, i: (0, i, 0)),
            pl.BlockSpec((NC, HH), lambda p, i: (0, 0)),
            pl.BlockSpec((NC, HH), lambda p, i: (0, 0)),
            pl.BlockSpec((NC, HH), lambda p, i: (0, 0)),
        ],
        out_specs=pl.BlockSpec((NC, RB, HH), lambda p, i: (0, i, 0)),
        out_shape=jax.ShapeDtypeStruct((NC, N_NODES, HH), jnp.float32),
        scratch_shapes=[
            pltpu.VMEM((NC, HH), jnp.float32),
            pltpu.VMEM((NC, HH), jnp.float32),
        ],
    )(agg, hwp, degp, b, g, be)


def _mlp_body(gp_ref, w1_ref, b1_ref, g1_ref, be1_ref,
              w2_ref, b2_ref, g2_ref, be2_ref, w3_ref, b3_ref, out_ref):
    gp = gp_ref[...]
    g = jnp.concatenate([gp[0], gp[1]], axis=1)

    def bn(z, ga, be):
        mu = jnp.mean(z, axis=0)
        d = z - mu
        var = jnp.mean(d * d, axis=0)
        return ga * (z - mu) / jnp.sqrt(var + BN_EPS) + be

    z = jnp.dot(g, w1_ref[...], preferred_element_type=jnp.float32) + b1_ref[...]
    z = jnp.maximum(bn(z, g1_ref[...], be1_ref[...]), 0.0)
    z = jnp.dot(z, w2_ref[...], preferred_element_type=jnp.float32) + b2_ref[...]
    z = jnp.maximum(bn(z, g2_ref[...], be2_ref[...]), 0.0)
    out_ref[...] = jnp.dot(z, w3_ref[...], preferred_element_type=jnp.float32) + b3_ref[...]


def _mlp(gparts, m):
    args = (gparts, m['W1'], m['b1'].reshape(1, -1), m['g1'].reshape(1, -1),
            m['be1'].reshape(1, -1), m['W2'], m['b2'].reshape(1, -1),
            m['g2'].reshape(1, -1), m['be2'].reshape(1, -1), m['W3'],
            m['b3'].reshape(1, -1))
    return pl.pallas_call(
        _mlp_body,
        out_shape=jax.ShapeDtypeStruct((NG, 2), jnp.float32),
    )(*args)


def kernel(x, edge_index, batch, params):
    ei = edge_index.astype(jnp.int32)
    src, dst = ei[0], ei[1]
    pad = E_PAD - N_EDGES
    srcp = jnp.concatenate([src, jnp.zeros((pad,), jnp.int32)])
    dstp = jnp.concatenate([dst, jnp.full((pad,), DUMMY, jnp.int32)])
    eidx = jnp.stack([srcp.reshape(NS, NCHT, ECH),
                      dstp.reshape(NS, NCHT, ECH)], axis=2)
    bat = batch.astype(jnp.int32)
    batT = bat[:TOFF].reshape(NS, NPJ, PCH)
    batTail = bat[TOFF:].reshape(1, TAIL)
    zeros = jnp.zeros((RPS, HH), jnp.float32)
    ones128 = jnp.ones((ECH, HH), jnp.float32)

    w1p = jnp.concatenate([params['gcn'][0]['W'],
                           jnp.zeros((KPAD - IN_DIM, HID), jnp.float32)], axis=0)

    degp = _sc_deg(eidx, ones128, zeros)
    hwp = _tca1(x.astype(jnp.int32), w1p, degp)
    h = None
    for li, lp in enumerate(params['gcn']):
        agg = _sc_agg(hwp, eidx, zeros)
        h = _tcb(agg, hwp, degp, lp['b'].reshape(NC, HH),
                 lp['gamma'].reshape(NC, HH), lp['beta'].reshape(NC, HH))
        if li < len(params['gcn']) - 1:
            hwp = _tca(h, params['gcn'][li + 1]['W'], degp)
    g = _sc_pool(h, batT, batTail, zeros)
    return _mlp(g, params['mlp'])


# speedup vs baseline: 8.3645x; 1.0339x over previous
import functools

import jax
import jax.numpy as jnp
import numpy as np
from jax import lax
from jax.experimental import pallas as pl
from jax.experimental.pallas import tpu as pltpu
from jax.experimental.pallas import tpu_sc as plsc

N_NODES = 10000
N_EDGES = 160000
HID = 256
HH = 128
NG = 512
NS = 16
NC = 2
ECH = 128
NCHT = 79
E_PAD = NS * NCHT * ECH
ACCN = 10008
DUMMY = 10000
RPS = 624
TAIL = N_NODES - NS * RPS
TOFF = NS * RPS
PCH = 104
NPJ = RPS // PCH
RB = 1000
NRB = N_NODES // RB
BN_EPS = 1e-5
ALPHA = 0.01
IN_DIM = 82
_FEATURE_LENS = [44, 7, 6, 7, 2, 2, 6, 8]

_mesh = plsc.VectorSubcoreMesh(core_axis_name="c", subcore_axis_name="s")


@functools.partial(
    pl.kernel,
    out_type=jax.ShapeDtypeStruct((NC, N_NODES, HH), jnp.float32),
    mesh=_mesh,
    scratch_types=[
        pltpu.VMEM((2, 2, ECH), jnp.int32),
        pltpu.VMEM((ECH, HH), jnp.float32),
        pltpu.VMEM_SHARED((ACCN, HH), jnp.float32),
        pltpu.SemaphoreType.DMA,
        pltpu.SemaphoreType.DMA,
    ],
)
def _sc_deg(eidx, ones, zeros, out, idx_v, ones_v, acc, semA, semB):
    c = lax.axis_index("c")
    s = lax.axis_index("s")
    pltpu.sync_copy(ones, ones_v)
    pltpu.sync_copy(zeros, acc.at[pl.ds(s * RPS, RPS)])

    @pl.when(s == 0)
    def _():
        pltpu.sync_copy(zeros.at[pl.ds(0, TAIL)], acc.at[pl.ds(TOFF, TAIL)])

    plsc.subcore_barrier()

    et = eidx.at[s]
    start = c * 40
    pltpu.async_copy(et.at[start], idx_v.at[0], semA)

    def step(t, carry):
        j = start + 2 * t
        pltpu.async_copy(et.at[j + 1], idx_v.at[1], semB)
        pltpu.make_async_copy(et.at[0], idx_v.at[0], semA).wait()
        pltpu.sync_copy(ones_v, acc.at[idx_v.at[0, 1]], add=True)
        pltpu.async_copy(et.at[jnp.minimum(j + 2, NCHT - 1)], idx_v.at[0], semA)
        pltpu.make_async_copy(et.at[0], idx_v.at[1], semB).wait()
        pltpu.sync_copy(ones_v, acc.at[idx_v.at[1, 1]], add=True)
        return carry

    lax.fori_loop(0, 20 - c, step, 0)
    pltpu.make_async_copy(et.at[0], idx_v.at[0], semA).wait()

    @pl.when(c == 1)
    def _():
        pltpu.sync_copy(ones_v, acc.at[idx_v.at[0, 1]], add=True)

    plsc.subcore_barrier()
    pltpu.sync_copy(acc.at[pl.ds(s * RPS, RPS)], out.at[c, pl.ds(s * RPS, RPS)])

    @pl.when(s == 0)
    def _():
        pltpu.sync_copy(acc.at[pl.ds(TOFF, TAIL)], out.at[c, pl.ds(TOFF, TAIL)])


@functools.partial(
    pl.kernel,
    out_type=jax.ShapeDtypeStruct((NC, N_NODES, HH), jnp.float32),
    mesh=_mesh,
    scratch_types=[
        pltpu.VMEM((4, 2, ECH), jnp.int32),
        pltpu.VMEM((ECH, HH), jnp.float32),
        pltpu.VMEM((ECH, HH), jnp.float32),
        pltpu.VMEM_SHARED((ACCN, HH), jnp.float32),
        pltpu.SemaphoreType.DMA,
        pltpu.SemaphoreType.DMA,
        pltpu.SemaphoreType.DMA,
        pltpu.SemaphoreType.DMA,
        pltpu.SemaphoreType.DMA,
        pltpu.SemaphoreType.DMA,
        pltpu.SemaphoreType.DMA,
        pltpu.SemaphoreType.DMA,
    ],
)
def _sc_agg(hwp, eidx, zeros, out, idx_v, buf0, buf1, acc,
            gs0, gs1, ss0, ss1, is0, is1, is2, is3):
    c = lax.axis_index("c")
    s = lax.axis_index("s")
    pltpu.sync_copy(zeros, acc.at[pl.ds(s * RPS, RPS)])

    @pl.when(s == 0)
    def _():
        pltpu.sync_copy(zeros.at[pl.ds(0, TAIL)], acc.at[pl.ds(TOFF, TAIL)])

    plsc.subcore_barrier()

    hw_c = hwp.at[c]
    et = eidx.at[s]
    isems = (is0, is1, is2, is3)
    bufs = (buf0, buf1)
    gsems = (gs0, gs1)
    ssems = (ss0, ss1)

    def fetch(jj, slot, sem):
        pltpu.async_copy(et.at[jj], idx_v.at[slot], sem)

    def wait_idx(slot, sem):
        pltpu.make_async_copy(et.at[0], idx_v.at[slot], sem).wait()

    def gather(slot, buf, sem):
        pltpu.async_copy(hw_c.at[idx_v.at[slot, 0]], buf, sem)

    def wait_gather(slot, buf, sem):
        pltpu.make_async_copy(hw_c.at[idx_v.at[slot, 0]], buf, sem).wait()

    def scatter(slot, buf, sem):
        pltpu.async_copy(buf, acc.at[idx_v.at[slot, 1]], sem, add=True)

    def wait_scatter(slot, buf, sem):
        pltpu.make_async_copy(buf, acc.at[idx_v.at[slot, 1]], sem).wait()

    fetch(0, 0, is0)
    fetch(1, 1, is1)
    wait_idx(0, is0)
    gather(0, buf0, gs0)
    fetch(2, 2, is2)
    wait_idx(1, is1)
    gather(1, buf1, gs1)
    fetch(3, 3, is3)
    wait_gather(0, buf0, gs0)
    scatter(0, buf0, ss0)
    wait_gather(1, buf1, gs1)
    scatter(1, buf1, ss1)

    def sub(jj, slot, fslot, b):
        wait_scatter(slot, bufs[b], ssems[b])
        fetch(jj, fslot, isems[fslot])
        wait_idx(slot, isems[slot])
        gather(slot, bufs[b], gsems[b])
        wait_gather(slot, bufs[b], gsems[b])
        scatter(slot, bufs[b], ssems[b])

    def quad(t, carry):
        j = 4 * t + 2
        sub(j + 2, 2, 0, 0)
        sub(j + 3, 3, 1, 1)
        sub(jnp.minimum(j + 4, NCHT - 1), 0, 2, 0)
        sub(jnp.minimum(j + 5, NCHT - 1), 1, 3, 1)
        return carry

    lax.fori_loop(0, 19, quad, 0)
    wait_scatter(2, buf0, ss0)
    wait_idx(2, is2)
    gather(2, buf0, gs0)
    wait_gather(2, buf0, gs0)
    scatter(2, buf0, ss0)
    wait_scatter(3, buf1, ss1)
    wait_scatter(2, buf0, ss0)
    wait_idx(3, is3)
    plsc.subcore_barrier()
    pltpu.sync_copy(acc.at[pl.ds(s * RPS, RPS)], out.at[c, pl.ds(s * RPS, RPS)])

    @pl.when(s == 0)
    def _():
        pltpu.sync_copy(acc.at[pl.ds(TOFF, TAIL)], out.at[c, pl.ds(TOFF, TAIL)])


@functools.partial(
    pl.kernel,
    out_type=jax.ShapeDtypeStruct((NC, NG, HH), jnp.float32),
    mesh=_mesh,
    scratch_types=[
        pltpu.VMEM((NPJ, PCH), jnp.int32),
        pltpu.VMEM((1, TAIL), jnp.int32),
        pltpu.VMEM((PCH, HH), jnp.float32),
        pltpu.VMEM_SHARED((NG, HH), jnp.float32),
    ],
)
def _sc_pool(h, batT, batTail, zeros, out, bat_v, batt_v, buf, acc):
    c = lax.axis_index("c")
    s = lax.axis_index("s")
    gps = NG // NS
    pltpu.sync_copy(batT.at[s], bat_v)
    pltpu.sync_copy(batTail, batt_v)
    pltpu.sync_copy(zeros.at[pl.ds(0, gps)], acc.at[pl.ds(s * gps, gps)])
    plsc.subcore_barrier()
    for j in range(NPJ):
        pltpu.sync_copy(h.at[c, pl.ds(s * RPS + j * PCH, PCH)], buf)
        pltpu.sync_copy(buf, acc.at[bat_v.at[j]], add=True)

    @pl.when(s == 0)
    def _():
        pltpu.sync_copy(h.at[c, pl.ds(TOFF, TAIL)], buf.at[pl.ds(0, TAIL)])
        pltpu.sync_copy(buf.at[pl.ds(0, TAIL)], acc.at[batt_v.at[0]], add=True)

    plsc.subcore_barrier()
    pltpu.sync_copy(acc.at[pl.ds(s * gps, gps)], out.at[c, pl.ds(s * gps, gps)])


KPAD = 128
_OFFS = [0, 44, 51, 57, 64, 66, 68, 74]


def _dinv_from(deg):
    return lax.rsqrt(deg[0, :, 0] + deg[1, :, 0] + 1.0)


def _tca1_body(x_ref, w_ref, deg_ref, out_ref):
    dinv = _dinv_from(deg_ref[...])
    xv = x_ref[...]
    cols = lax.broadcasted_iota(jnp.int32, (RB, KPAD), 1)
    oh = jnp.zeros((RB, KPAD), jnp.float32)
    for i, o in enumerate(_OFFS):
        oh = oh + (cols == (xv[:, i:i + 1] + o)).astype(jnp.float32)
    t = jnp.dot(oh, w_ref[...], preferred_element_type=jnp.float32)
    t = t * dinv[:, None]
    out_ref[0] = t[:, :HH]
    out_ref[1] = t[:, HH:]


def _tca1(x, w1p, degp):
    return pl.pallas_call(
        _tca1_body,
        grid=(NRB,),
        in_specs=[
            pl.BlockSpec((RB, 8), lambda i: (i, 0)),
            pl.BlockSpec((KPAD, HID), lambda i: (0, 0)),
            pl.BlockSpec((NC, RB, HH), lambda i: (0, i, 0)),
        ],
        out_specs=pl.BlockSpec((NC, RB, HH), lambda i: (0, i, 0)),
        out_shape=jax.ShapeDtypeStruct((NC, N_NODES, HH), jnp.float32),
    )(x, w1p, degp)


def _tca_body(h_ref, w_ref, deg_ref, out_ref):
    dinv = _dinv_from(deg_ref[...])
    h = h_ref[...]
    hh = jnp.concatenate([h[0], h[1]], axis=1)
    t = jnp.dot(hh, w_ref[...], preferred_element_type=jnp.float32)
    t = t * dinv[:, None]
    out_ref[0] = t[:, :HH]
    out_ref[1] = t[:, HH:]


def _tca(h, w, degp):
    return pl.pallas_call(
        _tca_body,
        grid=(NRB,),
        in_specs=[
            pl.BlockSpec((NC, RB, HH), lambda i: (0, i, 0)),
            pl.BlockSpec((HID, HID), lambda i: (0, 0)),
            pl.BlockSpec((NC, RB, HH), lambda i: (0, i, 0)),
        ],
        out_specs=pl.BlockSpec((NC, RB, HH), lambda i: (0, i, 0)),
        out_shape=jax.ShapeDtypeStruct((NC, N_NODES, HH), jnp.float32),
    )(h, w, degp)


def _tcb_body(agg_ref, hwp_ref, deg_ref, b_ref, g_ref, be_ref, out_ref, ssum, ssq):
    p = pl.program_id(0)
    i = pl.program_id(1)
    dinv = _dinv_from(deg_ref[...])[None, :, None]
    t = (agg_ref[...] + hwp_ref[...]) * dinv + b_ref[...][:, None, :]

    @pl.when((p == 0) & (i == 0))
    def _():
        ssum[...] = jnp.zeros_like(ssum)
        ssq[...] = jnp.zeros_like(ssq)

    @pl.when(p == 0)
    def _():
        ssum[...] += t.sum(axis=1)

    @pl.when(p == 1)
    def _():
        mu = (ssum[...] / N_NODES)[:, None, :]
        d = t - mu
        ssq[...] += (d * d).sum(axis=1)

    @pl.when(p == 2)
    def _():
        mu = (ssum[...] / N_NODES)[:, None, :]
        var = (ssq[...] / N_NODES)[:, None, :]
        z = g_ref[...][:, None, :] * (t - mu) / jnp.sqrt(var + BN_EPS) \
            + be_ref[...][:, None, :]
        out_ref[...] = jnp.where(z > 0, z, ALPHA * z)


def _tcb(agg, hwp, degp, b, g, be):
    return pl.pallas_call(
        _tcb_body,
        grid=(3, NRB),
        in_specs=[
            pl.BlockSpec((NC, RB, HH), lambda p, i: (0, i, 0)),
            pl.BlockSpec((NC, RB, HH), lambda p, i: (0, i, 0)),
            pl.BlockSpec((NC, RB, HH), lambda p, i: (0, i, 0)),
            pl.BlockSpec((NC, HH), lambda p, i: (0, 0)),
            pl.BlockSpec((NC, HH), lambda p, i: (0, 0)),
            pl.BlockSpec((NC, HH), lambda p, i: (0, 0)),
        ],
        out_specs=pl.BlockSpec((NC, RB, HH), lambda p, i: (0, i, 0)),
        out_shape=jax.ShapeDtypeStruct((NC, N_NODES, HH), jnp.float32),
        scratch_shapes=[
            pltpu.VMEM((NC, HH), jnp.float32),
            pltpu.VMEM((NC, HH), jnp.float32),
        ],
    )(agg, hwp, degp, b, g, be)


def _mlp_body(gp_ref, w1_ref, b1_ref, g1_ref, be1_ref,
              w2_ref, b2_ref, g2_ref, be2_ref, w3_ref, b3_ref, out_ref):
    gp = gp_ref[...]
    g = jnp.concatenate([gp[0], gp[1]], axis=1)

    def bn(z, ga, be):
        mu = jnp.mean(z, axis=0)
        d = z - mu
        var = jnp.mean(d * d, axis=0)
        return ga * (z - mu) / jnp.sqrt(var + BN_EPS) + be

    z = jnp.dot(g, w1_ref[...], preferred_element_type=jnp.float32) + b1_ref[...]
    z = jnp.maximum(bn(z, g1_ref[...], be1_ref[...]), 0.0)
    z = jnp.dot(z, w2_ref[...], preferred_element_type=jnp.float32) + b2_ref[...]
    z = jnp.maximum(bn(z, g2_ref[...], be2_ref[...]), 0.0)
    out_ref[...] = jnp.dot(z, w3_ref[...], preferred_element_type=jnp.float32) + b3_ref[...]


def _mlp(gparts, m):
    args = (gparts, m['W1'], m['b1'].reshape(1, -1), m['g1'].reshape(1, -1),
            m['be1'].reshape(1, -1), m['W2'], m['b2'].reshape(1, -1),
            m['g2'].reshape(1, -1), m['be2'].reshape(1, -1), m['W3'],
            m['b3'].reshape(1, -1))
    return pl.pallas_call(
        _mlp_body,
        out_shape=jax.ShapeDtypeStruct((NG, 2), jnp.float32),
    )(*args)


def kernel(x, edge_index, batch, params):
    ei = edge_index.astype(jnp.int32)
    src, dst = ei[0], ei[1]
    pad = E_PAD - N_EDGES
    srcp = jnp.concatenate([src, jnp.zeros((pad,), jnp.int32)])
    dstp = jnp.concatenate([dst, jnp.full((pad,), DUMMY, jnp.int32)])
    eidx = jnp.stack([srcp.reshape(NS, NCHT, ECH),
                      dstp.reshape(NS, NCHT, ECH)], axis=2)
    bat = batch.astype(jnp.int32)
    batT = bat[:TOFF].reshape(NS, NPJ, PCH)
    batTail = bat[TOFF:].reshape(1, TAIL)
    zeros = jnp.zeros((RPS, HH), jnp.float32)
    ones128 = jnp.ones((ECH, HH), jnp.float32)

    w1p = jnp.concatenate([params['gcn'][0]['W'],
                           jnp.zeros((KPAD - IN_DIM, HID), jnp.float32)], axis=0)

    degp = _sc_deg(eidx, ones128, zeros)
    hwp = _tca1(x.astype(jnp.int32), w1p, degp)
    h = None
    for li, lp in enumerate(params['gcn']):
        agg = _sc_agg(hwp, eidx, zeros)
        h = _tcb(agg, hwp, degp, lp['b'].reshape(NC, HH),
                 lp['gamma'].reshape(NC, HH), lp['beta'].reshape(NC, HH))
        if li < len(params['gcn']) - 1:
            hwp = _tca(h, params['gcn'][li + 1]['W'], degp)
    g = _sc_pool(h, batT, batTail, zeros)
    return _mlp(g, params['mlp'])


# speedup vs baseline: 8.9503x; 1.0700x over previous
import functools

import jax
import jax.numpy as jnp
import numpy as np
from jax import lax
from jax.experimental import pallas as pl
from jax.experimental.pallas import tpu as pltpu
from jax.experimental.pallas import tpu_sc as plsc

N_NODES = 10000
N_EDGES = 160000
HID = 256
HH = 128
NG = 512
NS = 16
NC = 2
ECH = 128
NCHT = 79
E_PAD = NS * NCHT * ECH
ACCN = 10008
DUMMY = 10000
RPS = 624
TAIL = N_NODES - NS * RPS
TOFF = NS * RPS
PCH = 104
NPJ = RPS // PCH
RB = 1000
NRB = N_NODES // RB
BN_EPS = 1e-5
ALPHA = 0.01
IN_DIM = 82
_FEATURE_LENS = [44, 7, 6, 7, 2, 2, 6, 8]

_mesh = plsc.VectorSubcoreMesh(core_axis_name="c", subcore_axis_name="s")


@functools.partial(
    pl.kernel,
    out_type=jax.ShapeDtypeStruct((NC, N_NODES, HH), jnp.float32),
    mesh=_mesh,
    scratch_types=[
        pltpu.VMEM((2, 2, ECH), jnp.int32),
        pltpu.VMEM((ECH, HH), jnp.float32),
        pltpu.VMEM_SHARED((ACCN, HH), jnp.float32),
        pltpu.SemaphoreType.DMA,
        pltpu.SemaphoreType.DMA,
    ],
)
def _sc_deg(eidx, ones, zeros, out, idx_v, ones_v, acc, semA, semB):
    c = lax.axis_index("c")
    s = lax.axis_index("s")
    pltpu.sync_copy(ones, ones_v)
    pltpu.sync_copy(zeros, acc.at[pl.ds(s * RPS, RPS)])

    @pl.when(s == 0)
    def _():
        pltpu.sync_copy(zeros.at[pl.ds(0, TAIL)], acc.at[pl.ds(TOFF, TAIL)])

    plsc.subcore_barrier()

    et = eidx.at[s]
    start = c * 40
    pltpu.async_copy(et.at[start], idx_v.at[0], semA)

    def step(t, carry):
        j = start + 2 * t
        pltpu.async_copy(et.at[j + 1], idx_v.at[1], semB)
        pltpu.make_async_copy(et.at[0], idx_v.at[0], semA).wait()
        pltpu.sync_copy(ones_v, acc.at[idx_v.at[0, 1]], add=True)
        pltpu.async_copy(et.at[jnp.minimum(j + 2, NCHT - 1)], idx_v.at[0], semA)
        pltpu.make_async_copy(et.at[0], idx_v.at[1], semB).wait()
        pltpu.sync_copy(ones_v, acc.at[idx_v.at[1, 1]], add=True)
        return carry

    lax.fori_loop(0, 20 - c, step, 0)
    pltpu.make_async_copy(et.at[0], idx_v.at[0], semA).wait()

    @pl.when(c == 1)
    def _():
        pltpu.sync_copy(ones_v, acc.at[idx_v.at[0, 1]], add=True)

    plsc.subcore_barrier()
    pltpu.sync_copy(acc.at[pl.ds(s * RPS, RPS)], out.at[c, pl.ds(s * RPS, RPS)])

    @pl.when(s == 0)
    def _():
        pltpu.sync_copy(acc.at[pl.ds(TOFF, TAIL)], out.at[c, pl.ds(TOFF, TAIL)])


@functools.partial(
    pl.kernel,
    out_type=jax.ShapeDtypeStruct((NC, N_NODES, HH), jnp.float32),
    mesh=_mesh,
    scratch_types=[
        pltpu.VMEM((4, 2, ECH), jnp.int32),
        pltpu.VMEM((ECH, HH), jnp.float32),
        pltpu.VMEM((ECH, HH), jnp.float32),
        pltpu.VMEM_SHARED((ACCN, HH), jnp.float32),
        pltpu.SemaphoreType.DMA,
        pltpu.SemaphoreType.DMA,
        pltpu.SemaphoreType.DMA,
        pltpu.SemaphoreType.DMA,
        pltpu.SemaphoreType.DMA,
        pltpu.SemaphoreType.DMA,
        pltpu.SemaphoreType.DMA,
        pltpu.SemaphoreType.DMA,
        pltpu.SemaphoreType.DMA,
        pltpu.SemaphoreType.DMA,
    ],
)
def _sc_agg(hwp, eidx, zeros, out, idx_v, buf0, buf1, acc,
            g00, g01, g10, g11, ss0, ss1, is0, is1, is2, is3):
    c = lax.axis_index("c")
    s = lax.axis_index("s")
    pltpu.sync_copy(zeros, acc.at[pl.ds(s * RPS, RPS)])

    @pl.when(s == 0)
    def _():
        pltpu.sync_copy(zeros.at[pl.ds(0, TAIL)], acc.at[pl.ds(TOFF, TAIL)])

    plsc.subcore_barrier()

    hw_c = hwp.at[c]
    et = eidx.at[s]
    isems = (is0, is1, is2, is3)
    bufs = (buf0, buf1)
    gsems = ((g00, g01), (g10, g11))
    ssems = (ss0, ss1)
    EH = ECH // 2

    def fetch(jj, slot):
        pltpu.async_copy(et.at[jj], idx_v.at[slot], isems[slot])

    def wait_idx(slot):
        pltpu.make_async_copy(et.at[0], idx_v.at[slot], isems[slot]).wait()

    def gather2(slot, b):
        buf = bufs[b]
        pltpu.async_copy(hw_c.at[idx_v.at[slot, 0, pl.ds(0, EH)]],
                         buf.at[pl.ds(0, EH)], gsems[b][0])
        pltpu.async_copy(hw_c.at[idx_v.at[slot, 0, pl.ds(EH, EH)]],
                         buf.at[pl.ds(EH, EH)], gsems[b][1])

    def wait_gather2(slot, b):
        buf = bufs[b]
        pltpu.make_async_copy(hw_c.at[idx_v.at[slot, 0, pl.ds(0, EH)]],
                              buf.at[pl.ds(0, EH)], gsems[b][0]).wait()
        pltpu.make_async_copy(hw_c.at[idx_v.at[slot, 0, pl.ds(EH, EH)]],
                              buf.at[pl.ds(EH, EH)], gsems[b][1]).wait()

    def scatter(slot, b):
        pltpu.async_copy(bufs[b], acc.at[idx_v.at[slot, 1]], ssems[b], add=True)

    def wait_scatter(slot, b):
        pltpu.make_async_copy(bufs[b], acc.at[idx_v.at[slot, 1]], ssems[b]).wait()

    fetch(0, 0)
    fetch(1, 1)
    wait_idx(0)
    gather2(0, 0)
    fetch(2, 2)
    wait_idx(1)
    gather2(1, 1)
    fetch(3, 3)
    wait_gather2(0, 0)
    scatter(0, 0)

    def sub(jj, slot, fslot, b):
        wait_scatter(slot, b)
        fetch(jj, fslot)
        wait_idx(slot)
        gather2(slot, b)
        wait_gather2((slot + 3) % 4, 1 - b)
        scatter((slot + 3) % 4, 1 - b)

    def quad(t, carry):
        k = 4 * t + 2
        sub(k + 2, 2, 0, 0)
        sub(k + 3, 3, 1, 1)
        sub(jnp.minimum(k + 4, NCHT - 1), 0, 2, 0)
        sub(jnp.minimum(k + 5, NCHT - 1), 1, 3, 1)
        return carry

    lax.fori_loop(0, 19, quad, 0)
    wait_scatter(2, 0)
    wait_idx(2)
    gather2(2, 0)
    wait_gather2(1, 1)
    scatter(1, 1)
    wait_gather2(2, 0)
    scatter(2, 0)
    wait_scatter(3, 1)
    wait_scatter(2, 0)
    wait_idx(3)
    plsc.subcore_barrier()
    pltpu.sync_copy(acc.at[pl.ds(s * RPS, RPS)], out.at[c, pl.ds(s * RPS, RPS)])

    @pl.when(s == 0)
    def _():
        pltpu.sync_copy(acc.at[pl.ds(TOFF, TAIL)], out.at[c, pl.ds(TOFF, TAIL)])


@functools.partial(
    pl.kernel,
    out_type=jax.ShapeDtypeStruct((NC, NG, HH), jnp.float32),
    mesh=_mesh,
    scratch_types=[
        pltpu.VMEM((NPJ, PCH), jnp.int32),
        pltpu.VMEM((1, TAIL), jnp.int32),
        pltpu.VMEM((PCH, HH), jnp.float32),
        pltpu.VMEM_SHARED((NG, HH), jnp.float32),
    ],
)
def _sc_pool(h, batT, batTail, zeros, out, bat_v, batt_v, buf, acc):
    c = lax.axis_index("c")
    s = lax.axis_index("s")
    gps = NG // NS
    pltpu.sync_copy(batT.at[s], bat_v)
    pltpu.sync_copy(batTail, batt_v)
    pltpu.sync_copy(zeros.at[pl.ds(0, gps)], acc.at[pl.ds(s * gps, gps)])
    plsc.subcore_barrier()
    for j in range(NPJ):
        pltpu.sync_copy(h.at[c, pl.ds(s * RPS + j * PCH, PCH)], buf)
        pltpu.sync_copy(buf, acc.at[bat_v.at[j]], add=True)

    @pl.when(s == 0)
    def _():
        pltpu.sync_copy(h.at[c, pl.ds(TOFF, TAIL)], buf.at[pl.ds(0, TAIL)])
        pltpu.sync_copy(buf.at[pl.ds(0, TAIL)], acc.at[batt_v.at[0]], add=True)

    plsc.subcore_barrier()
    pltpu.sync_copy(acc.at[pl.ds(s * gps, gps)], out.at[c, pl.ds(s * gps, gps)])


KPAD = 128
_OFFS = [0, 44, 51, 57, 64, 66, 68, 74]


def _dinv_from(deg):
    return lax.rsqrt(deg[0, :, 0] + deg[1, :, 0] + 1.0)


def _tca1_body(x_ref, w_ref, deg_ref, out_ref):
    dinv = _dinv_from(deg_ref[...])
    xv = x_ref[...]
    cols = lax.broadcasted_iota(jnp.int32, (RB, KPAD), 1)
    oh = jnp.zeros((RB, KPAD), jnp.float32)
    for i, o in enumerate(_OFFS):
        oh = oh + (cols == (xv[:, i:i + 1] + o)).astype(jnp.float32)
    t = jnp.dot(oh, w_ref[...], preferred_element_type=jnp.float32)
    t = t * dinv[:, None]
    out_ref[0] = t[:, :HH]
    out_ref[1] = t[:, HH:]


def _tca1(x, w1p, degp):
    return pl.pallas_call(
        _tca1_body,
        grid=(NRB,),
        in_specs=[
            pl.BlockSpec((RB, 8), lambda i: (i, 0)),
            pl.BlockSpec((KPAD, HID), lambda i: (0, 0)),
            pl.BlockSpec((NC, RB, HH), lambda i: (0, i, 0)),
        ],
        out_specs=pl.BlockSpec((NC, RB, HH), lambda i: (0, i, 0)),
        out_shape=jax.ShapeDtypeStruct((NC, N_NODES, HH), jnp.float32),
    )(x, w1p, degp)


def _tca_body(h_ref, w_ref, deg_ref, out_ref):
    dinv = _dinv_from(deg_ref[...])
    h = h_ref[...]
    hh = jnp.concatenate([h[0], h[1]], axis=1)
    t = jnp.dot(hh, w_ref[...], preferred_element_type=jnp.float32)
    t = t * dinv[:, None]
    out_ref[0] = t[:, :HH]
    out_ref[1] = t[:, HH:]


def _tca(h, w, degp):
    return pl.pallas_call(
        _tca_body,
        grid=(NRB,),
        in_specs=[
            pl.BlockSpec((NC, RB, HH), lambda i: (0, i, 0)),
            pl.BlockSpec((HID, HID), lambda i: (0, 0)),
            pl.BlockSpec((NC, RB, HH), lambda i: (0, i, 0)),
        ],
        out_specs=pl.BlockSpec((NC, RB, HH), lambda i: (0, i, 0)),
        out_shape=jax.ShapeDtypeStruct((NC, N_NODES, HH), jnp.float32),
    )(h, w, degp)


def _tcb_body(agg_ref, hwp_ref, deg_ref, b_ref, g_ref, be_ref, out_ref, ssum, ssq):
    p = pl.program_id(0)
    i = pl.program_id(1)
    dinv = _dinv_from(deg_ref[...])[None, :, None]
    t = (agg_ref[...] + hwp_ref[...]) * dinv + b_ref[...][:, None, :]

    @pl.when((p == 0) & (i == 0))
    def _():
        ssum[...] = jnp.zeros_like(ssum)
        ssq[...] = jnp.zeros_like(ssq)

    @pl.when(p == 0)
    def _():
        ssum[...] += t.sum(axis=1)

    @pl.when(p == 1)
    def _():
        mu = (ssum[...] / N_NODES)[:, None, :]
        d = t - mu
        ssq[...] += (d * d).sum(axis=1)

    @pl.when(p == 2)
    def _():
        mu = (ssum[...] / N_NODES)[:, None, :]
        var = (ssq[...] / N_NODES)[:, None, :]
        z = g_ref[...][:, None, :] * (t - mu) / jnp.sqrt(var + BN_EPS) \
            + be_ref[...][:, None, :]
        out_ref[...] = jnp.where(z > 0, z, ALPHA * z)


def _tcb(agg, hwp, degp, b, g, be):
    return pl.pallas_call(
        _tcb_body,
        grid=(3, NRB),
        in_specs=[
            pl.BlockSpec((NC, RB, HH), lambda p, i: (0, i, 0)),
            pl.BlockSpec((NC, RB, HH), lambda p, i: (0, i, 0)),
            pl.BlockSpec((NC, RB, HH), lambda p, i: (0, i, 0)),
            pl.BlockSpec((NC, HH), lambda p, i: (0, 0)),
            pl.BlockSpec((NC, HH), lambda p, i: (0, 0)),
            pl.BlockSpec((NC, HH), lambda p, i: (0, 0)),
        ],
        out_specs=pl.BlockSpec((NC, RB, HH), lambda p, i: (0, i, 0)),
        out_shape=jax.ShapeDtypeStruct((NC, N_NODES, HH), jnp.float32),
        scratch_shapes=[
            pltpu.VMEM((NC, HH), jnp.float32),
            pltpu.VMEM((NC, HH), jnp.float32),
        ],
    )(agg, hwp, degp, b, g, be)


def _mlp_body(gp_ref, w1_ref, b1_ref, g1_ref, be1_ref,
              w2_ref, b2_ref, g2_ref, be2_ref, w3_ref, b3_ref, out_ref):
    gp = gp_ref[...]
    g = jnp.concatenate([gp[0], gp[1]], axis=1)

    def bn(z, ga, be):
        mu = jnp.mean(z, axis=0)
        d = z - mu
        var = jnp.mean(d * d, axis=0)
        return ga * (z - mu) / jnp.sqrt(var + BN_EPS) + be

    z = jnp.dot(g, w1_ref[...], preferred_element_type=jnp.float32) + b1_ref[...]
    z = jnp.maximum(bn(z, g1_ref[...], be1_ref[...]), 0.0)
    z = jnp.dot(z, w2_ref[...], preferred_element_type=jnp.float32) + b2_ref[...]
    z = jnp.maximum(bn(z, g2_ref[...], be2_ref[...]), 0.0)
    out_ref[...] = jnp.dot(z, w3_ref[...], preferred_element_type=jnp.float32) + b3_ref[...]


def _mlp(gparts, m):
    args = (gparts, m['W1'], m['b1'].reshape(1, -1), m['g1'].reshape(1, -1),
            m['be1'].reshape(1, -1), m['W2'], m['b2'].reshape(1, -1),
            m['g2'].reshape(1, -1), m['be2'].reshape(1, -1), m['W3'],
            m['b3'].reshape(1, -1))
    return pl.pallas_call(
        _mlp_body,
        out_shape=jax.ShapeDtypeStruct((NG, 2), jnp.float32),
    )(*args)


def kernel(x, edge_index, batch, params):
    ei = edge_index.astype(jnp.int32)
    src, dst = ei[0], ei[1]
    pad = E_PAD - N_EDGES
    srcp = jnp.concatenate([src, jnp.zeros((pad,), jnp.int32)])
    dstp = jnp.concatenate([dst, jnp.full((pad,), DUMMY, jnp.int32)])
    eidx = jnp.stack([srcp.reshape(NS, NCHT, ECH),
                      dstp.reshape(NS, NCHT, ECH)], axis=2)
    bat = batch.astype(jnp.int32)
    batT = bat[:TOFF].reshape(NS, NPJ, PCH)
    batTail = bat[TOFF:].reshape(1, TAIL)
    zeros = jnp.zeros((RPS, HH), jnp.float32)
    ones128 = jnp.ones((ECH, HH), jnp.float32)

    w1p = jnp.concatenate([params['gcn'][0]['W'],
                           jnp.zeros((KPAD - IN_DIM, HID), jnp.float32)], axis=0)

    degp = _sc_deg(eidx, ones128, zeros)
    hwp = _tca1(x.astype(jnp.int32), w1p, degp)
    h = None
    for li, lp in enumerate(params['gcn']):
        agg = _sc_agg(hwp, eidx, zeros)
        h = _tcb(agg, hwp, degp, lp['b'].reshape(NC, HH),
                 lp['gamma'].reshape(NC, HH), lp['beta'].reshape(NC, HH))
        if li < len(params['gcn']) - 1:
            hwp = _tca(h, params['gcn'][li + 1]['W'], degp)
    g = _sc_pool(h, batT, batTail, zeros)
    return _mlp(g, params['mlp'])


# speedup vs baseline: 9.1978x; 1.0277x over previous
import functools

import jax
import jax.numpy as jnp
import numpy as np
from jax import lax
from jax.experimental import pallas as pl
from jax.experimental.pallas import tpu as pltpu
from jax.experimental.pallas import tpu_sc as plsc

N_NODES = 10000
N_EDGES = 160000
HID = 256
HH = 128
NG = 512
NS = 16
NC = 2
ECH = 128
NCHT = 79
E_PAD = NS * NCHT * ECH
ACCN = 10008
DUMMY = 10000
RPS = 624
TAIL = N_NODES - NS * RPS
TOFF = NS * RPS
PCH = 104
NPJ = RPS // PCH
RB = 1000
NRB = N_NODES // RB
BN_EPS = 1e-5
ALPHA = 0.01
IN_DIM = 82
_FEATURE_LENS = [44, 7, 6, 7, 2, 2, 6, 8]

_mesh = plsc.VectorSubcoreMesh(core_axis_name="c", subcore_axis_name="s")


@functools.partial(
    pl.kernel,
    out_type=jax.ShapeDtypeStruct((NC, N_NODES, HH), jnp.float32),
    mesh=_mesh,
    scratch_types=[
        pltpu.VMEM((2, 2, ECH), jnp.int32),
        pltpu.VMEM((ECH, HH), jnp.float32),
        pltpu.VMEM_SHARED((ACCN, HH), jnp.float32),
        pltpu.SemaphoreType.DMA,
        pltpu.SemaphoreType.DMA,
    ],
)
def _sc_deg(eidx, ones, zeros, out, idx_v, ones_v, acc, semA, semB):
    c = lax.axis_index("c")
    s = lax.axis_index("s")
    pltpu.sync_copy(ones, ones_v)
    pltpu.sync_copy(zeros, acc.at[pl.ds(s * RPS, RPS)])

    @pl.when(s == 0)
    def _():
        pltpu.sync_copy(zeros.at[pl.ds(0, TAIL)], acc.at[pl.ds(TOFF, TAIL)])

    plsc.subcore_barrier()

    et = eidx.at[s]
    start = c * 40
    pltpu.async_copy(et.at[start], idx_v.at[0], semA)

    def step(t, carry):
        j = start + 2 * t
        pltpu.async_copy(et.at[j + 1], idx_v.at[1], semB)
        pltpu.make_async_copy(et.at[0], idx_v.at[0], semA).wait()
        pltpu.sync_copy(ones_v, acc.at[idx_v.at[0, 1]], add=True)
        pltpu.async_copy(et.at[jnp.minimum(j + 2, NCHT - 1)], idx_v.at[0], semA)
        pltpu.make_async_copy(et.at[0], idx_v.at[1], semB).wait()
        pltpu.sync_copy(ones_v, acc.at[idx_v.at[1, 1]], add=True)
        return carry

    lax.fori_loop(0, 20 - c, step, 0)
    pltpu.make_async_copy(et.at[0], idx_v.at[0], semA).wait()

    @pl.when(c == 1)
    def _():
        pltpu.sync_copy(ones_v, acc.at[idx_v.at[0, 1]], add=True)

    plsc.subcore_barrier()
    pltpu.sync_copy(acc.at[pl.ds(s * RPS, RPS)], out.at[c, pl.ds(s * RPS, RPS)])

    @pl.when(s == 0)
    def _():
        pltpu.sync_copy(acc.at[pl.ds(TOFF, TAIL)], out.at[c, pl.ds(TOFF, TAIL)])


@functools.partial(
    pl.kernel,
    out_type=jax.ShapeDtypeStruct((NC, N_NODES, HH), jnp.float32),
    mesh=_mesh,
    scratch_types=[
        pltpu.VMEM((4, 2, ECH), jnp.int32),
        pltpu.VMEM((ECH, HH), jnp.float32),
        pltpu.VMEM((ECH, HH), jnp.float32),
        pltpu.VMEM_SHARED((ACCN, HH), jnp.float32),
        pltpu.SemaphoreType.DMA,
        pltpu.SemaphoreType.DMA,
        pltpu.SemaphoreType.DMA,
        pltpu.SemaphoreType.DMA,
        pltpu.SemaphoreType.DMA,
        pltpu.SemaphoreType.DMA,
        pltpu.SemaphoreType.DMA,
        pltpu.SemaphoreType.DMA,
        pltpu.SemaphoreType.DMA,
        pltpu.SemaphoreType.DMA,
    ],
)
def _sc_agg(hwp, eidx, zeros, out, idx_v, buf0, buf1, acc,
            g00, g01, g10, g11, ss0, ss1, is0, is1, is2, is3):
    c = lax.axis_index("c")
    s = lax.axis_index("s")
    pltpu.sync_copy(zeros, acc.at[pl.ds(s * RPS, RPS)])

    @pl.when(s == 0)
    def _():
        pltpu.sync_copy(zeros.at[pl.ds(0, TAIL)], acc.at[pl.ds(TOFF, TAIL)])

    plsc.subcore_barrier()

    hw_c = hwp.at[c]
    et = eidx.at[s]
    isems = (is0, is1, is2, is3)
    bufs = (buf0, buf1)
    gsems = ((g00, g01), (g10, g11))
    ssems = (ss0, ss1)
    EH = ECH // 2

    def fetch(jj, slot):
        pltpu.async_copy(et.at[jj], idx_v.at[slot], isems[slot])

    def wait_idx(slot):
        pltpu.make_async_copy(et.at[0], idx_v.at[slot], isems[slot]).wait()

    def gather2(slot, b):
        buf = bufs[b]
        pltpu.async_copy(hw_c.at[idx_v.at[slot, 0, pl.ds(0, EH)]],
                         buf.at[pl.ds(0, EH)], gsems[b][0])
        pltpu.async_copy(hw_c.at[idx_v.at[slot, 0, pl.ds(EH, EH)]],
                         buf.at[pl.ds(EH, EH)], gsems[b][1])

    def wait_gather2(slot, b):
        buf = bufs[b]
        pltpu.make_async_copy(hw_c.at[idx_v.at[slot, 0, pl.ds(0, EH)]],
                              buf.at[pl.ds(0, EH)], gsems[b][0]).wait()
        pltpu.make_async_copy(hw_c.at[idx_v.at[slot, 0, pl.ds(EH, EH)]],
                              buf.at[pl.ds(EH, EH)], gsems[b][1]).wait()

    def scatter(slot, b):
        pltpu.async_copy(bufs[b], acc.at[idx_v.at[slot, 1]], ssems[b], add=True)

    def wait_scatter(slot, b):
        pltpu.make_async_copy(bufs[b], acc.at[idx_v.at[slot, 1]], ssems[b]).wait()

    fetch(0, 0)
    fetch(1, 1)
    wait_idx(0)
    gather2(0, 0)
    fetch(2, 2)
    wait_idx(1)
    gather2(1, 1)
    fetch(3, 3)
    wait_gather2(0, 0)
    scatter(0, 0)

    def sub(jj, slot, fslot, b):
        wait_scatter(slot, b)
        fetch(jj, fslot)
        wait_idx(slot)
        gather2(slot, b)
        wait_gather2((slot + 3) % 4, 1 - b)
        scatter((slot + 3) % 4, 1 - b)

    def quad(t, carry):
        k = 4 * t + 2
        sub(k + 2, 2, 0, 0)
        sub(k + 3, 3, 1, 1)
        sub(jnp.minimum(k + 4, NCHT - 1), 0, 2, 0)
        sub(jnp.minimum(k + 5, NCHT - 1), 1, 3, 1)
        return carry

    lax.fori_loop(0, 19, quad, 0)
    wait_scatter(2, 0)
    wait_idx(2)
    gather2(2, 0)
    wait_gather2(1, 1)
    scatter(1, 1)
    wait_gather2(2, 0)
    scatter(2, 0)
    wait_scatter(3, 1)
    wait_scatter(2, 0)
    wait_idx(3)
    plsc.subcore_barrier()
    pltpu.sync_copy(acc.at[pl.ds(s * RPS, RPS)], out.at[c, pl.ds(s * RPS, RPS)])

    @pl.when(s == 0)
    def _():
        pltpu.sync_copy(acc.at[pl.ds(TOFF, TAIL)], out.at[c, pl.ds(TOFF, TAIL)])


@functools.partial(
    pl.kernel,
    out_type=jax.ShapeDtypeStruct((NC, NG, HH), jnp.float32),
    mesh=_mesh,
    scratch_types=[
        pltpu.VMEM((NPJ, PCH), jnp.int32),
        pltpu.VMEM((1, TAIL), jnp.int32),
        pltpu.VMEM((PCH, HH), jnp.float32),
        pltpu.VMEM_SHARED((NG, HH), jnp.float32),
    ],
)
def _sc_pool(h, batT, batTail, zeros, out, bat_v, batt_v, buf, acc):
    c = lax.axis_index("c")
    s = lax.axis_index("s")
    gps = NG // NS
    pltpu.sync_copy(batT.at[s], bat_v)
    pltpu.sync_copy(batTail, batt_v)
    pltpu.sync_copy(zeros.at[pl.ds(0, gps)], acc.at[pl.ds(s * gps, gps)])
    plsc.subcore_barrier()
    for j in range(NPJ):
        pltpu.sync_copy(h.at[c, pl.ds(s * RPS + j * PCH, PCH)], buf)
        pltpu.sync_copy(buf, acc.at[bat_v.at[j]], add=True)

    @pl.when(s == 0)
    def _():
        pltpu.sync_copy(h.at[c, pl.ds(TOFF, TAIL)], buf.at[pl.ds(0, TAIL)])
        pltpu.sync_copy(buf.at[pl.ds(0, TAIL)], acc.at[batt_v.at[0]], add=True)

    plsc.subcore_barrier()
    pltpu.sync_copy(acc.at[pl.ds(s * gps, gps)], out.at[c, pl.ds(s * gps, gps)])


KPAD = 128
_OFFS = [0, 44, 51, 57, 64, 66, 68, 74]


def _dinv_from(deg):
    return lax.rsqrt(deg[0, :, 0] + deg[1, :, 0] + 1.0)


def _tca1_body(x_ref, w_ref, deg_ref, out_ref):
    dinv = _dinv_from(deg_ref[...])
    xv = x_ref[...]
    cols = lax.broadcasted_iota(jnp.int32, (RB, KPAD), 1)
    oh = jnp.zeros((RB, KPAD), jnp.float32)
    for i, o in enumerate(_OFFS):
        oh = oh + (cols == (xv[:, i:i + 1] + o)).astype(jnp.float32)
    t = jnp.dot(oh, w_ref[...], preferred_element_type=jnp.float32)
    t = t * dinv[:, None]
    out_ref[0] = t[:, :HH]
    out_ref[1] = t[:, HH:]


def _tca1(x, w1p, degp):
    return pl.pallas_call(
        _tca1_body,
        grid=(NRB,),
        in_specs=[
            pl.BlockSpec((RB, 8), lambda i: (i, 0)),
            pl.BlockSpec((KPAD, HID), lambda i: (0, 0)),
            pl.BlockSpec((NC, RB, HH), lambda i: (0, i, 0)),
        ],
        out_specs=pl.BlockSpec((NC, RB, HH), lambda i: (0, i, 0)),
        out_shape=jax.ShapeDtypeStruct((NC, N_NODES, HH), jnp.float32),
    )(x, w1p, degp)


def _tca_body(h_ref, w_ref, deg_ref, out_ref):
    dinv = _dinv_from(deg_ref[...])
    h = h_ref[...]
    hh = jnp.concatenate([h[0], h[1]], axis=1)
    t = jnp.dot(hh, w_ref[...], preferred_element_type=jnp.float32)
    t = t * dinv[:, None]
    out_ref[0] = t[:, :HH]
    out_ref[1] = t[:, HH:]


def _tca(h, w, degp):
    return pl.pallas_call(
        _tca_body,
        grid=(NRB,),
        in_specs=[
            pl.BlockSpec((NC, RB, HH), lambda i: (0, i, 0)),
            pl.BlockSpec((HID, HID), lambda i: (0, 0)),
            pl.BlockSpec((NC, RB, HH), lambda i: (0, i, 0)),
        ],
        out_specs=pl.BlockSpec((NC, RB, HH), lambda i: (0, i, 0)),
        out_shape=jax.ShapeDtypeStruct((NC, N_NODES, HH), jnp.float32),
    )(h, w, degp)


def _tcb_body(agg_ref, hwp_ref, deg_ref, b_ref, g_ref, be_ref, out_ref, ssum, ssq):
    p = pl.program_id(0)
    i = pl.program_id(1)
    dinv = _dinv_from(deg_ref[...])[None, :, None]
    t = (agg_ref[...] + hwp_ref[...]) * dinv + b_ref[...][:, None, :]

    @pl.when((p == 0) & (i == 0))
    def _():
        ssum[...] = jnp.zeros_like(ssum)
        ssq[...] = jnp.zeros_like(ssq)

    @pl.when(p == 0)
    def _():
        ssum[...] += t.sum(axis=1)

    @pl.when(p == 1)
    def _():
        mu = (ssum[...] / N_NODES)[:, None, :]
        d = t - mu
        ssq[...] += (d * d).sum(axis=1)

    @pl.when(p == 2)
    def _():
        mu = (ssum[...] / N_NODES)[:, None, :]
        var = (ssq[...] / N_NODES)[:, None, :]
        z = g_ref[...][:, None, :] * (t - mu) / jnp.sqrt(var + BN_EPS) \
            + be_ref[...][:, None, :]
        out_ref[...] = jnp.where(z > 0, z, ALPHA * z)


def _tcb(agg, hwp, degp, b, g, be):
    return pl.pallas_call(
        _tcb_body,
        grid=(3, NRB),
        in_specs=[
            pl.BlockSpec((NC, RB, HH), lambda p, i: (0, i, 0)),
            pl.BlockSpec((NC, RB, HH), lambda p, i: (0, i, 0)),
            pl.BlockSpec((NC, RB, HH), lambda p, i: (0, i, 0)),
            pl.BlockSpec((NC, HH), lambda p, i: (0, 0)),
            pl.BlockSpec((NC, HH), lambda p, i: (0, 0)),
            pl.BlockSpec((NC, HH), lambda p, i: (0, 0)),
        ],
        out_specs=pl.BlockSpec((NC, RB, HH), lambda p, i: (0, i, 0)),
        out_shape=jax.ShapeDtypeStruct((NC, N_NODES, HH), jnp.float32),
        scratch_shapes=[
            pltpu.VMEM((NC, HH), jnp.float32),
            pltpu.VMEM((NC, HH), jnp.float32),
        ],
    )(agg, hwp, degp, b, g, be)


def _tcbw_body(agg_ref, hwp_ref, deg_ref, b_ref, g_ref, be_ref, w_ref,
               out_ref, ssum, ssq):
    p = pl.program_id(0)
    i = pl.program_id(1)
    dinv1 = _dinv_from(deg_ref[...])
    dinv = dinv1[None, :, None]
    t = (agg_ref[...] + hwp_ref[...]) * dinv + b_ref[...][:, None, :]

    @pl.when((p == 0) & (i == 0))
    def _():
        ssum[...] = jnp.zeros_like(ssum)
        ssq[...] = jnp.zeros_like(ssq)

    @pl.when(p == 0)
    def _():
        ssum[...] += t.sum(axis=1)

    @pl.when(p == 1)
    def _():
        mu = (ssum[...] / N_NODES)[:, None, :]
        d = t - mu
        ssq[...] += (d * d).sum(axis=1)

    @pl.when(p == 2)
    def _():
        mu = (ssum[...] / N_NODES)[:, None, :]
        var = (ssq[...] / N_NODES)[:, None, :]
        z = g_ref[...][:, None, :] * (t - mu) / jnp.sqrt(var + BN_EPS) \
            + be_ref[...][:, None, :]
        h = jnp.where(z > 0, z, ALPHA * z)
        hh = jnp.concatenate([h[0], h[1]], axis=1)
        t2 = jnp.dot(hh, w_ref[...], preferred_element_type=jnp.float32)
        t2 = t2 * dinv1[:, None]
        out_ref[0] = t2[:, :HH]
        out_ref[1] = t2[:, HH:]


def _tcbw(agg, hwp, degp, b, g, be, w):
    return pl.pallas_call(
        _tcbw_body,
        grid=(3, NRB),
        in_specs=[
            pl.BlockSpec((NC, RB, HH), lambda p, i: (0, i, 0)),
            pl.BlockSpec((NC, RB, HH), lambda p, i: (0, i, 0)),
            pl.BlockSpec((NC, RB, HH), lambda p, i: (0, i, 0)),
            pl.BlockSpec((NC, HH), lambda p, i: (0, 0)),
            pl.BlockSpec((NC, HH), lambda p, i: (0, 0)),
            pl.BlockSpec((NC, HH), lambda p, i: (0, 0)),
            pl.BlockSpec((HID, HID), lambda p, i: (0, 0)),
        ],
        out_specs=pl.BlockSpec((NC, RB, HH), lambda p, i: (0, i, 0)),
        out_shape=jax.ShapeDtypeStruct((NC, N_NODES, HH), jnp.float32),
        scratch_shapes=[
            pltpu.VMEM((NC, HH), jnp.float32),
            pltpu.VMEM((NC, HH), jnp.float32),
        ],
    )(agg, hwp, degp, b, g, be, w)


def _mlp_body(gp_ref, w1_ref, b1_ref, g1_ref, be1_ref,
              w2_ref, b2_ref, g2_ref, be2_ref, w3_ref, b3_ref, out_ref):
    gp = gp_ref[...]
    g = jnp.concatenate([gp[0], gp[1]], axis=1)

    def bn(z, ga, be):
        mu = jnp.mean(z, axis=0)
        d = z - mu
        var = jnp.mean(d * d, axis=0)
        return ga * (z - mu) / jnp.sqrt(var + BN_EPS) + be

    z = jnp.dot(g, w1_ref[...], preferred_element_type=jnp.float32) + b1_ref[...]
    z = jnp.maximum(bn(z, g1_ref[...], be1_ref[...]), 0.0)
    z = jnp.dot(z, w2_ref[...], preferred_element_type=jnp.float32) + b2_ref[...]
    z = jnp.maximum(bn(z, g2_ref[...], be2_ref[...]), 0.0)
    out_ref[...] = jnp.dot(z, w3_ref[...], preferred_element_type=jnp.float32) + b3_ref[...]


def _mlp(gparts, m):
    args = (gparts, m['W1'], m['b1'].reshape(1, -1), m['g1'].reshape(1, -1),
            m['be1'].reshape(1, -1), m['W2'], m['b2'].reshape(1, -1),
            m['g2'].reshape(1, -1), m['be2'].reshape(1, -1), m['W3'],
            m['b3'].reshape(1, -1))
    return pl.pallas_call(
        _mlp_body,
        out_shape=jax.ShapeDtypeStruct((NG, 2), jnp.float32),
    )(*args)


def kernel(x, edge_index, batch, params):
    ei = edge_index.astype(jnp.int32)
    src, dst = ei[0], ei[1]
    pad = E_PAD - N_EDGES
    srcp = jnp.concatenate([src, jnp.zeros((pad,), jnp.int32)])
    dstp = jnp.concatenate([dst, jnp.full((pad,), DUMMY, jnp.int32)])
    eidx = jnp.stack([srcp.reshape(NS, NCHT, ECH),
                      dstp.reshape(NS, NCHT, ECH)], axis=2)
    bat = batch.astype(jnp.int32)
    batT = bat[:TOFF].reshape(NS, NPJ, PCH)
    batTail = bat[TOFF:].reshape(1, TAIL)
    zeros = jnp.zeros((RPS, HH), jnp.float32)
    ones128 = jnp.ones((ECH, HH), jnp.float32)

    w1p = jnp.concatenate([params['gcn'][0]['W'],
                           jnp.zeros((KPAD - IN_DIM, HID), jnp.float32)], axis=0)

    degp = _sc_deg(eidx, ones128, zeros)
    hwp = _tca1(x.astype(jnp.int32), w1p, degp)
    h = None
    for li, lp in enumerate(params['gcn']):
        agg = _sc_agg(hwp, eidx, zeros)
        bb = lp['b'].reshape(NC, HH)
        gg = lp['gamma'].reshape(NC, HH)
        be = lp['beta'].reshape(NC, HH)
        if li < len(params['gcn']) - 1:
            hwp = _tcbw(agg, hwp, degp, bb, gg, be, params['gcn'][li + 1]['W'])
        else:
            h = _tcb(agg, hwp, degp, bb, gg, be)
    g = _sc_pool(h, batT, batTail, zeros)
    return _mlp(g, params['mlp'])


# speedup vs baseline: 9.1991x; 1.0001x over previous
import functools

import jax
import jax.numpy as jnp
import numpy as np
from jax import lax
from jax.experimental import pallas as pl
from jax.experimental.pallas import tpu as pltpu
from jax.experimental.pallas import tpu_sc as plsc

N_NODES = 10000
N_EDGES = 160000
HID = 256
HH = 128
NG = 512
NS = 16
NC = 2
ECH = 128
NCHT = 79
E_PAD = NS * NCHT * ECH
ACCN = 10008
DUMMY = 10000
RPS = 624
TAIL = N_NODES - NS * RPS
TOFF = NS * RPS
PCH = 104
NPJ = RPS // PCH
RB = 1000
NRB = N_NODES // RB
BN_EPS = 1e-5
ALPHA = 0.01
IN_DIM = 82
_FEATURE_LENS = [44, 7, 6, 7, 2, 2, 6, 8]

_mesh = plsc.VectorSubcoreMesh(core_axis_name="c", subcore_axis_name="s")


@functools.partial(
    pl.kernel,
    out_type=jax.ShapeDtypeStruct((NC, N_NODES, HH), jnp.float32),
    mesh=_mesh,
    scratch_types=[
        pltpu.VMEM((2, 2, ECH), jnp.int32),
        pltpu.VMEM((ECH, HH), jnp.float32),
        pltpu.VMEM_SHARED((ACCN, HH), jnp.float32),
        pltpu.SemaphoreType.DMA,
        pltpu.SemaphoreType.DMA,
    ],
)
def _sc_deg(eidx, ones, zeros, out, idx_v, ones_v, acc, semA, semB):
    c = lax.axis_index("c")
    s = lax.axis_index("s")
    pltpu.sync_copy(ones, ones_v)
    pltpu.sync_copy(zeros, acc.at[pl.ds(s * RPS, RPS)])

    @pl.when(s == 0)
    def _():
        pltpu.sync_copy(zeros.at[pl.ds(0, TAIL)], acc.at[pl.ds(TOFF, TAIL)])

    plsc.subcore_barrier()

    et = eidx.at[s]
    start = c * 40
    pltpu.async_copy(et.at[start], idx_v.at[0], semA)

    def step(t, carry):
        j = start + 2 * t
        pltpu.async_copy(et.at[j + 1], idx_v.at[1], semB)
        pltpu.make_async_copy(et.at[0], idx_v.at[0], semA).wait()
        pltpu.sync_copy(ones_v, acc.at[idx_v.at[0, 1]], add=True)
        pltpu.async_copy(et.at[jnp.minimum(j + 2, NCHT - 1)], idx_v.at[0], semA)
        pltpu.make_async_copy(et.at[0], idx_v.at[1], semB).wait()
        pltpu.sync_copy(ones_v, acc.at[idx_v.at[1, 1]], add=True)
        return carry

    lax.fori_loop(0, 20 - c, step, 0)
    pltpu.make_async_copy(et.at[0], idx_v.at[0], semA).wait()

    @pl.when(c == 1)
    def _():
        pltpu.sync_copy(ones_v, acc.at[idx_v.at[0, 1]], add=True)

    plsc.subcore_barrier()
    pltpu.sync_copy(acc.at[pl.ds(s * RPS, RPS)], out.at[c, pl.ds(s * RPS, RPS)])

    @pl.when(s == 0)
    def _():
        pltpu.sync_copy(acc.at[pl.ds(TOFF, TAIL)], out.at[c, pl.ds(TOFF, TAIL)])


@functools.partial(
    pl.kernel,
    out_type=jax.ShapeDtypeStruct((NC, N_NODES, HH), jnp.float32),
    mesh=_mesh,
    scratch_types=[
        pltpu.VMEM((4, 2, ECH), jnp.int32),
        pltpu.VMEM((ECH, HH), jnp.float32),
        pltpu.VMEM((ECH, HH), jnp.float32),
        pltpu.VMEM_SHARED((ACCN, HH), jnp.float32),
        pltpu.SemaphoreType.DMA,
        pltpu.SemaphoreType.DMA,
        pltpu.SemaphoreType.DMA,
        pltpu.SemaphoreType.DMA,
        pltpu.SemaphoreType.DMA,
        pltpu.SemaphoreType.DMA,
        pltpu.SemaphoreType.DMA,
        pltpu.SemaphoreType.DMA,
        pltpu.SemaphoreType.DMA,
        pltpu.SemaphoreType.DMA,
    ],
)
def _sc_agg(hwp, eidx, zeros, out, idx_v, buf0, buf1, acc,
            g00, g01, g10, g11, ss0, ss1, is0, is1, is2, is3):
    c = lax.axis_index("c")
    s = lax.axis_index("s")
    pltpu.sync_copy(zeros, acc.at[pl.ds(s * RPS, RPS)])

    @pl.when(s == 0)
    def _():
        pltpu.sync_copy(zeros.at[pl.ds(0, TAIL)], acc.at[pl.ds(TOFF, TAIL)])

    plsc.subcore_barrier()

    hw_c = hwp.at[c]
    et = eidx.at[s]
    isems = (is0, is1, is2, is3)
    bufs = (buf0, buf1)
    gsems = ((g00, g01), (g10, g11))
    ssems = (ss0, ss1)
    EH = ECH // 2

    def fetch(jj, slot):
        pltpu.async_copy(et.at[jj], idx_v.at[slot], isems[slot])

    def wait_idx(slot):
        pltpu.make_async_copy(et.at[0], idx_v.at[slot], isems[slot]).wait()

    def gather2(slot, b):
        buf = bufs[b]
        pltpu.async_copy(hw_c.at[idx_v.at[slot, 0, pl.ds(0, EH)]],
                         buf.at[pl.ds(0, EH)], gsems[b][0])
        pltpu.async_copy(hw_c.at[idx_v.at[slot, 0, pl.ds(EH, EH)]],
                         buf.at[pl.ds(EH, EH)], gsems[b][1])

    def wait_gather2(slot, b):
        buf = bufs[b]
        pltpu.make_async_copy(hw_c.at[idx_v.at[slot, 0, pl.ds(0, EH)]],
                              buf.at[pl.ds(0, EH)], gsems[b][0]).wait()
        pltpu.make_async_copy(hw_c.at[idx_v.at[slot, 0, pl.ds(EH, EH)]],
                              buf.at[pl.ds(EH, EH)], gsems[b][1]).wait()

    def scatter(slot, b):
        pltpu.async_copy(bufs[b], acc.at[idx_v.at[slot, 1]], ssems[b], add=True)

    def wait_scatter(slot, b):
        pltpu.make_async_copy(bufs[b], acc.at[idx_v.at[slot, 1]], ssems[b]).wait()

    fetch(0, 0)
    fetch(1, 1)
    wait_idx(0)
    gather2(0, 0)
    fetch(2, 2)
    wait_idx(1)
    gather2(1, 1)
    fetch(3, 3)
    wait_gather2(0, 0)
    scatter(0, 0)

    def sub(jj, slot, fslot, b):
        wait_scatter(slot, b)
        fetch(jj, fslot)
        wait_idx(slot)
        gather2(slot, b)
        wait_gather2((slot + 3) % 4, 1 - b)
        scatter((slot + 3) % 4, 1 - b)

    def quad(t, carry):
        k = 4 * t + 2
        sub(k + 2, 2, 0, 0)
        sub(k + 3, 3, 1, 1)
        sub(jnp.minimum(k + 4, NCHT - 1), 0, 2, 0)
        sub(jnp.minimum(k + 5, NCHT - 1), 1, 3, 1)
        return carry

    lax.fori_loop(0, 19, quad, 0)
    wait_scatter(2, 0)
    wait_idx(2)
    gather2(2, 0)
    wait_gather2(1, 1)
    scatter(1, 1)
    wait_gather2(2, 0)
    scatter(2, 0)
    wait_scatter(3, 1)
    wait_scatter(2, 0)
    wait_idx(3)
    plsc.subcore_barrier()
    pltpu.sync_copy(acc.at[pl.ds(s * RPS, RPS)], out.at[c, pl.ds(s * RPS, RPS)])

    @pl.when(s == 0)
    def _():
        pltpu.sync_copy(acc.at[pl.ds(TOFF, TAIL)], out.at[c, pl.ds(TOFF, TAIL)])


@functools.partial(
    pl.kernel,
    out_type=jax.ShapeDtypeStruct((NC, NG, HH), jnp.float32),
    mesh=_mesh,
    scratch_types=[
        pltpu.VMEM((NPJ, PCH), jnp.int32),
        pltpu.VMEM((1, TAIL), jnp.int32),
        pltpu.VMEM((PCH, HH), jnp.float32),
        pltpu.VMEM_SHARED((NG, HH), jnp.float32),
    ],
)
def _sc_pool(h, batT, batTail, zeros, out, bat_v, batt_v, buf, acc):
    c = lax.axis_index("c")
    s = lax.axis_index("s")
    gps = NG // NS
    pltpu.sync_copy(batT.at[s], bat_v)
    pltpu.sync_copy(batTail, batt_v)
    pltpu.sync_copy(zeros.at[pl.ds(0, gps)], acc.at[pl.ds(s * gps, gps)])
    plsc.subcore_barrier()
    for j in range(NPJ):
        pltpu.sync_copy(h.at[c, pl.ds(s * RPS + j * PCH, PCH)], buf)
        pltpu.sync_copy(buf, acc.at[bat_v.at[j]], add=True)

    @pl.when(s == 0)
    def _():
        pltpu.sync_copy(h.at[c, pl.ds(TOFF, TAIL)], buf.at[pl.ds(0, TAIL)])
        pltpu.sync_copy(buf.at[pl.ds(0, TAIL)], acc.at[batt_v.at[0]], add=True)

    plsc.subcore_barrier()
    pltpu.sync_copy(acc.at[pl.ds(s * gps, gps)], out.at[c, pl.ds(s * gps, gps)])


KPAD = 128
_OFFS = [0, 44, 51, 57, 64, 66, 68, 74]


def _dinv_from(deg):
    return lax.rsqrt(deg[0, :, 0] + deg[1, :, 0] + 1.0)


def _tca1_body(x_ref, w_ref, deg_ref, out_ref):
    dinv = _dinv_from(deg_ref[...])
    xv = x_ref[...]
    cols = lax.broadcasted_iota(jnp.int32, (RB, KPAD), 1)
    oh = jnp.zeros((RB, KPAD), jnp.float32)
    for i, o in enumerate(_OFFS):
        oh = oh + (cols == (xv[:, i:i + 1] + o)).astype(jnp.float32)
    t = jnp.dot(oh, w_ref[...], preferred_element_type=jnp.float32)
    t = t * dinv[:, None]
    out_ref[0] = t[:, :HH]
    out_ref[1] = t[:, HH:]


def _tca1(x, w1p, degp):
    return pl.pallas_call(
        _tca1_body,
        grid=(NRB,),
        in_specs=[
            pl.BlockSpec((RB, 8), lambda i: (i, 0)),
            pl.BlockSpec((KPAD, HID), lambda i: (0, 0)),
            pl.BlockSpec((NC, RB, 8), lambda i: (0, i, 0)),
        ],
        out_specs=pl.BlockSpec((NC, RB, HH), lambda i: (0, i, 0)),
        out_shape=jax.ShapeDtypeStruct((NC, N_NODES, HH), jnp.float32),
    )(x, w1p, degp)


def _tca_body(h_ref, w_ref, deg_ref, out_ref):
    dinv = _dinv_from(deg_ref[...])
    h = h_ref[...]
    hh = jnp.concatenate([h[0], h[1]], axis=1)
    t = jnp.dot(hh, w_ref[...], preferred_element_type=jnp.float32)
    t = t * dinv[:, None]
    out_ref[0] = t[:, :HH]
    out_ref[1] = t[:, HH:]


def _tca(h, w, degp):
    return pl.pallas_call(
        _tca_body,
        grid=(NRB,),
        in_specs=[
            pl.BlockSpec((NC, RB, HH), lambda i: (0, i, 0)),
            pl.BlockSpec((HID, HID), lambda i: (0, 0)),
            pl.BlockSpec((NC, RB, 8), lambda i: (0, i, 0)),
        ],
        out_specs=pl.BlockSpec((NC, RB, HH), lambda i: (0, i, 0)),
        out_shape=jax.ShapeDtypeStruct((NC, N_NODES, HH), jnp.float32),
    )(h, w, degp)


def _tcb_body(agg_ref, hwp_ref, deg_ref, b_ref, g_ref, be_ref, out_ref, ssum, ssq):
    p = pl.program_id(0)
    i = pl.program_id(1)
    dinv = _dinv_from(deg_ref[...])[None, :, None]
    t = (agg_ref[...] + hwp_ref[...]) * dinv + b_ref[...][:, None, :]

    @pl.when((p == 0) & (i == 0))
    def _():
        ssum[...] = jnp.zeros_like(ssum)
        ssq[...] = jnp.zeros_like(ssq)

    @pl.when(p == 0)
    def _():
        ssum[...] += t.sum(axis=1)

    @pl.when(p == 1)
    def _():
        mu = (ssum[...] / N_NODES)[:, None, :]
        d = t - mu
        ssq[...] += (d * d).sum(axis=1)

    @pl.when(p == 2)
    def _():
        mu = (ssum[...] / N_NODES)[:, None, :]
        var = (ssq[...] / N_NODES)[:, None, :]
        z = g_ref[...][:, None, :] * (t - mu) / jnp.sqrt(var + BN_EPS) \
            + be_ref[...][:, None, :]
        out_ref[...] = jnp.where(z > 0, z, ALPHA * z)


def _tcb(agg, hwp, degp, b, g, be):
    return pl.pallas_call(
        _tcb_body,
        grid=(3, NRB),
        in_specs=[
            pl.BlockSpec((NC, RB, HH), lambda p, i: (0, i, 0)),
            pl.BlockSpec((NC, RB, HH), lambda p, i: (0, i, 0)),
            pl.BlockSpec((NC, RB, 8), lambda p, i: (0, i, 0)),
            pl.BlockSpec((NC, HH), lambda p, i: (0, 0)),
            pl.BlockSpec((NC, HH), lambda p, i: (0, 0)),
            pl.BlockSpec((NC, HH), lambda p, i: (0, 0)),
        ],
        out_specs=pl.BlockSpec((NC, RB, HH), lambda p, i: (0, i, 0)),
        out_shape=jax.ShapeDtypeStruct((NC, N_NODES, HH), jnp.float32),
        scratch_shapes=[
            pltpu.VMEM((NC, HH), jnp.float32),
            pltpu.VMEM((NC, HH), jnp.float32),
        ],
    )(agg, hwp, degp, b, g, be)


def _tcbw_body(agg_ref, hwp_ref, deg_ref, b_ref, g_ref, be_ref, w_ref,
               out_ref, ssum, ssq):
    p = pl.program_id(0)
    i = pl.program_id(1)
    dinv1 = _dinv_from(deg_ref[...])
    dinv = dinv1[None, :, None]
    t = (agg_ref[...] + hwp_ref[...]) * dinv + b_ref[...][:, None, :]

    @pl.when((p == 0) & (i == 0))
    def _():
        ssum[...] = jnp.zeros_like(ssum)
        ssq[...] = jnp.zeros_like(ssq)

    @pl.when(p == 0)
    def _():
        ssum[...] += t.sum(axis=1)

    @pl.when(p == 1)
    def _():
        mu = (ssum[...] / N_NODES)[:, None, :]
        d = t - mu
        ssq[...] += (d * d).sum(axis=1)

    @pl.when(p == 2)
    def _():
        mu = (ssum[...] / N_NODES)[:, None, :]
        var = (ssq[...] / N_NODES)[:, None, :]
        z = g_ref[...][:, None, :] * (t - mu) / jnp.sqrt(var + BN_EPS) \
            + be_ref[...][:, None, :]
        h = jnp.where(z > 0, z, ALPHA * z)
        hh = jnp.concatenate([h[0], h[1]], axis=1)
        t2 = jnp.dot(hh, w_ref[...], preferred_element_type=jnp.float32)
        t2 = t2 * dinv1[:, None]
        out_ref[0] = t2[:, :HH]
        out_ref[1] = t2[:, HH:]


def _tcbw(agg, hwp, degp, b, g, be, w):
    return pl.pallas_call(
        _tcbw_body,
        grid=(3, NRB),
        in_specs=[
            pl.BlockSpec((NC, RB, HH), lambda p, i: (0, i, 0)),
            pl.BlockSpec((NC, RB, HH), lambda p, i: (0, i, 0)),
            pl.BlockSpec((NC, RB, 8), lambda p, i: (0, i, 0)),
            pl.BlockSpec((NC, HH), lambda p, i: (0, 0)),
            pl.BlockSpec((NC, HH), lambda p, i: (0, 0)),
            pl.BlockSpec((NC, HH), lambda p, i: (0, 0)),
            pl.BlockSpec((HID, HID), lambda p, i: (0, 0)),
        ],
        out_specs=pl.BlockSpec((NC, RB, HH), lambda p, i: (0, i, 0)),
        out_shape=jax.ShapeDtypeStruct((NC, N_NODES, HH), jnp.float32),
        scratch_shapes=[
            pltpu.VMEM((NC, HH), jnp.float32),
            pltpu.VMEM((NC, HH), jnp.float32),
        ],
    )(agg, hwp, degp, b, g, be, w)


def _mlp_body(gp_ref, w1_ref, b1_ref, g1_ref, be1_ref,
              w2_ref, b2_ref, g2_ref, be2_ref, w3_ref, b3_ref, out_ref):
    gp = gp_ref[...]
    g = jnp.concatenate([gp[0], gp[1]], axis=1)

    def bn(z, ga, be):
        mu = jnp.mean(z, axis=0)
        d = z - mu
        var = jnp.mean(d * d, axis=0)
        return ga * (z - mu) / jnp.sqrt(var + BN_EPS) + be

    z = jnp.dot(g, w1_ref[...], preferred_element_type=jnp.float32) + b1_ref[...]
    z = jnp.maximum(bn(z, g1_ref[...], be1_ref[...]), 0.0)
    z = jnp.dot(z, w2_ref[...], preferred_element_type=jnp.float32) + b2_ref[...]
    z = jnp.maximum(bn(z, g2_ref[...], be2_ref[...]), 0.0)
    out_ref[...] = jnp.dot(z, w3_ref[...], preferred_element_type=jnp.float32) + b3_ref[...]


def _mlp(gparts, m):
    args = (gparts, m['W1'], m['b1'].reshape(1, -1), m['g1'].reshape(1, -1),
            m['be1'].reshape(1, -1), m['W2'], m['b2'].reshape(1, -1),
            m['g2'].reshape(1, -1), m['be2'].reshape(1, -1), m['W3'],
            m['b3'].reshape(1, -1))
    return pl.pallas_call(
        _mlp_body,
        out_shape=jax.ShapeDtypeStruct((NG, 2), jnp.float32),
    )(*args)


def kernel(x, edge_index, batch, params):
    ei = edge_index.astype(jnp.int32)
    src, dst = ei[0], ei[1]
    pad = E_PAD - N_EDGES
    srcp = jnp.concatenate([src, jnp.zeros((pad,), jnp.int32)])
    dstp = jnp.concatenate([dst, jnp.full((pad,), DUMMY, jnp.int32)])
    eidx = jnp.stack([srcp.reshape(NS, NCHT, ECH),
                      dstp.reshape(NS, NCHT, ECH)], axis=2)
    bat = batch.astype(jnp.int32)
    batT = bat[:TOFF].reshape(NS, NPJ, PCH)
    batTail = bat[TOFF:].reshape(1, TAIL)
    zeros = jnp.zeros((RPS, HH), jnp.float32)
    ones128 = jnp.ones((ECH, HH), jnp.float32)

    w1p = jnp.concatenate([params['gcn'][0]['W'],
                           jnp.zeros((KPAD - IN_DIM, HID), jnp.float32)], axis=0)

    degp = _sc_deg(eidx, ones128, zeros)
    degp = degp[:, :, :8]
    hwp = _tca1(x.astype(jnp.int32), w1p, degp)
    h = None
    for li, lp in enumerate(params['gcn']):
        agg = _sc_agg(hwp, eidx, zeros)
        bb = lp['b'].reshape(NC, HH)
        gg = lp['gamma'].reshape(NC, HH)
        be = lp['beta'].reshape(NC, HH)
        if li < len(params['gcn']) - 1:
            hwp = _tcbw(agg, hwp, degp, bb, gg, be, params['gcn'][li + 1]['W'])
        else:
            h = _tcb(agg, hwp, degp, bb, gg, be)
    g = _sc_pool(h, batT, batTail, zeros)
    return _mlp(g, params['mlp'])


# speedup vs baseline: 9.2044x; 1.0006x over previous
import functools

import jax
import jax.numpy as jnp
import numpy as np
from jax import lax
from jax.experimental import pallas as pl
from jax.experimental.pallas import tpu as pltpu
from jax.experimental.pallas import tpu_sc as plsc

N_NODES = 10000
N_EDGES = 160000
HID = 256
HH = 128
NG = 512
NS = 16
NC = 2
ECH = 128
NCHT = 79
E_PAD = NS * NCHT * ECH
ACCN = 10008
DUMMY = 10000
RPS = 624
TAIL = N_NODES - NS * RPS
TOFF = NS * RPS
PCH = 104
NPJ = RPS // PCH
RB = 1000
NRB = N_NODES // RB
BN_EPS = 1e-5
ALPHA = 0.01
IN_DIM = 82
_FEATURE_LENS = [44, 7, 6, 7, 2, 2, 6, 8]

_mesh = plsc.VectorSubcoreMesh(core_axis_name="c", subcore_axis_name="s")


@functools.partial(
    pl.kernel,
    out_type=jax.ShapeDtypeStruct((NC, N_NODES, HH), jnp.float32),
    mesh=_mesh,
    scratch_types=[
        pltpu.VMEM((2, 2, ECH), jnp.int32),
        pltpu.VMEM((ECH, HH), jnp.float32),
        pltpu.VMEM_SHARED((ACCN, HH), jnp.float32),
        pltpu.SemaphoreType.DMA,
        pltpu.SemaphoreType.DMA,
    ],
)
def _sc_deg(eidx, ones, zeros, out, idx_v, ones_v, acc, semA, semB):
    c = lax.axis_index("c")
    s = lax.axis_index("s")
    pltpu.sync_copy(ones, ones_v)
    pltpu.sync_copy(zeros, acc.at[pl.ds(s * RPS, RPS)])

    @pl.when(s == 0)
    def _():
        pltpu.sync_copy(zeros.at[pl.ds(0, TAIL)], acc.at[pl.ds(TOFF, TAIL)])

    plsc.subcore_barrier()

    et = eidx.at[s]
    start = c * 40
    pltpu.async_copy(et.at[start], idx_v.at[0], semA)

    def step(t, carry):
        j = start + 2 * t
        pltpu.async_copy(et.at[j + 1], idx_v.at[1], semB)
        pltpu.make_async_copy(et.at[0], idx_v.at[0], semA).wait()
        pltpu.sync_copy(ones_v, acc.at[idx_v.at[0, 1]], add=True)
        pltpu.async_copy(et.at[jnp.minimum(j + 2, NCHT - 1)], idx_v.at[0], semA)
        pltpu.make_async_copy(et.at[0], idx_v.at[1], semB).wait()
        pltpu.sync_copy(ones_v, acc.at[idx_v.at[1, 1]], add=True)
        return carry

    lax.fori_loop(0, 20 - c, step, 0)
    pltpu.make_async_copy(et.at[0], idx_v.at[0], semA).wait()

    @pl.when(c == 1)
    def _():
        pltpu.sync_copy(ones_v, acc.at[idx_v.at[0, 1]], add=True)

    plsc.subcore_barrier()
    pltpu.sync_copy(acc.at[pl.ds(s * RPS, RPS)], out.at[c, pl.ds(s * RPS, RPS)])

    @pl.when(s == 0)
    def _():
        pltpu.sync_copy(acc.at[pl.ds(TOFF, TAIL)], out.at[c, pl.ds(TOFF, TAIL)])


@functools.partial(
    pl.kernel,
    out_type=jax.ShapeDtypeStruct((NC, N_NODES, HH), jnp.float32),
    mesh=_mesh,
    scratch_types=[
        pltpu.VMEM((4, 2, ECH), jnp.int32),
        pltpu.VMEM((ECH, HH), jnp.float32),
        pltpu.VMEM((ECH, HH), jnp.float32),
        pltpu.VMEM_SHARED((ACCN, HH), jnp.float32),
        pltpu.SemaphoreType.DMA,
        pltpu.SemaphoreType.DMA,
        pltpu.SemaphoreType.DMA,
        pltpu.SemaphoreType.DMA,
        pltpu.SemaphoreType.DMA,
        pltpu.SemaphoreType.DMA,
        pltpu.SemaphoreType.DMA,
        pltpu.SemaphoreType.DMA,
        pltpu.SemaphoreType.DMA,
        pltpu.SemaphoreType.DMA,
    ],
)
def _sc_agg(hwp, eidx, zeros, out, idx_v, buf0, buf1, acc,
            g00, g01, g10, g11, ss0, ss1, is0, is1, is2, is3):
    c = lax.axis_index("c")
    s = lax.axis_index("s")
    pltpu.sync_copy(zeros, acc.at[pl.ds(s * RPS, RPS)])

    @pl.when(s == 0)
    def _():
        pltpu.sync_copy(zeros.at[pl.ds(0, TAIL)], acc.at[pl.ds(TOFF, TAIL)])

    plsc.subcore_barrier()

    hw_c = hwp.at[c]
    et = eidx.at[s]
    isems = (is0, is1, is2, is3)
    bufs = (buf0, buf1)
    gsems = ((g00, g01), (g10, g11))
    ssems = (ss0, ss1)
    EH = ECH // 2

    def fetch(jj, slot):
        pltpu.async_copy(et.at[jj], idx_v.at[slot], isems[slot])

    def wait_idx(slot):
        pltpu.make_async_copy(et.at[0], idx_v.at[slot], isems[slot]).wait()

    def gather2(slot, b):
        buf = bufs[b]
        pltpu.async_copy(hw_c.at[idx_v.at[slot, 0, pl.ds(0, EH)]],
                         buf.at[pl.ds(0, EH)], gsems[b][0])
        pltpu.async_copy(hw_c.at[idx_v.at[slot, 0, pl.ds(EH, EH)]],
                         buf.at[pl.ds(EH, EH)], gsems[b][1])

    def wait_gather2(slot, b):
        buf = bufs[b]
        pltpu.make_async_copy(hw_c.at[idx_v.at[slot, 0, pl.ds(0, EH)]],
                              buf.at[pl.ds(0, EH)], gsems[b][0]).wait()
        pltpu.make_async_copy(hw_c.at[idx_v.at[slot, 0, pl.ds(EH, EH)]],
                              buf.at[pl.ds(EH, EH)], gsems[b][1]).wait()

    def scatter(slot, b):
        pltpu.async_copy(bufs[b], acc.at[idx_v.at[slot, 1]], ssems[b], add=True)

    def wait_scatter(slot, b):
        pltpu.make_async_copy(bufs[b], acc.at[idx_v.at[slot, 1]], ssems[b]).wait()

    fetch(0, 0)
    fetch(1, 1)
    wait_idx(0)
    gather2(0, 0)
    fetch(2, 2)
    wait_idx(1)
    gather2(1, 1)
    fetch(3, 3)
    wait_gather2(0, 0)
    scatter(0, 0)

    def sub(jj, slot, fslot, b):
        wait_scatter(slot, b)
        fetch(jj, fslot)
        wait_idx(slot)
        gather2(slot, b)
        wait_gather2((slot + 3) % 4, 1 - b)
        scatter((slot + 3) % 4, 1 - b)

    def quad(t, carry):
        k = 4 * t + 2
        sub(k + 2, 2, 0, 0)
        sub(k + 3, 3, 1, 1)
        sub(jnp.minimum(k + 4, NCHT - 1), 0, 2, 0)
        sub(jnp.minimum(k + 5, NCHT - 1), 1, 3, 1)
        return carry

    lax.fori_loop(0, 19, quad, 0)
    wait_scatter(2, 0)
    wait_idx(2)
    gather2(2, 0)
    wait_gather2(1, 1)
    scatter(1, 1)
    wait_gather2(2, 0)
    scatter(2, 0)
    wait_scatter(3, 1)
    wait_scatter(2, 0)
    wait_idx(3)
    plsc.subcore_barrier()
    pltpu.sync_copy(acc.at[pl.ds(s * RPS, RPS)], out.at[c, pl.ds(s * RPS, RPS)])

    @pl.when(s == 0)
    def _():
        pltpu.sync_copy(acc.at[pl.ds(TOFF, TAIL)], out.at[c, pl.ds(TOFF, TAIL)])


@functools.partial(
    pl.kernel,
    out_type=jax.ShapeDtypeStruct((NC, NG, HH), jnp.float32),
    mesh=_mesh,
    scratch_types=[
        pltpu.VMEM((NPJ, PCH), jnp.int32),
        pltpu.VMEM((1, TAIL), jnp.int32),
        pltpu.VMEM((PCH, HH), jnp.float32),
        pltpu.VMEM_SHARED((NG, HH), jnp.float32),
    ],
)
def _sc_pool(h, batT, batTail, zeros, out, bat_v, batt_v, buf, acc):
    c = lax.axis_index("c")
    s = lax.axis_index("s")
    gps = NG // NS
    pltpu.sync_copy(batT.at[s], bat_v)
    pltpu.sync_copy(batTail, batt_v)
    pltpu.sync_copy(zeros.at[pl.ds(0, gps)], acc.at[pl.ds(s * gps, gps)])
    plsc.subcore_barrier()
    for j in range(NPJ):
        pltpu.sync_copy(h.at[c, pl.ds(s * RPS + j * PCH, PCH)], buf)
        pltpu.sync_copy(buf, acc.at[bat_v.at[j]], add=True)

    @pl.when(s == 0)
    def _():
        pltpu.sync_copy(h.at[c, pl.ds(TOFF, TAIL)], buf.at[pl.ds(0, TAIL)])
        pltpu.sync_copy(buf.at[pl.ds(0, TAIL)], acc.at[batt_v.at[0]], add=True)

    plsc.subcore_barrier()
    pltpu.sync_copy(acc.at[pl.ds(s * gps, gps)], out.at[c, pl.ds(s * gps, gps)])


KPAD = 128
_OFFS = [0, 44, 51, 57, 64, 66, 68, 74]


def _dinv_from(deg):
    return lax.rsqrt(deg[0, :, 0] + deg[1, :, 0] + 1.0)


def _tca1_body(x_ref, w_ref, deg_ref, out_ref):
    dinv = _dinv_from(deg_ref[...])
    xv = x_ref[...]
    cols = lax.broadcasted_iota(jnp.int32, (RB, KPAD), 1)
    oh = jnp.zeros((RB, KPAD), jnp.float32)
    for i, o in enumerate(_OFFS):
        oh = oh + (cols == (xv[:, i:i + 1] + o)).astype(jnp.float32)
    t = jnp.dot(oh, w_ref[...], preferred_element_type=jnp.float32)
    t = t * dinv[:, None]
    out_ref[0] = t[:, :HH]
    out_ref[1] = t[:, HH:]


def _tca1(x, w1p, degp):
    return pl.pallas_call(
        _tca1_body,
        grid=(NRB,),
        in_specs=[
            pl.BlockSpec((RB, 8), lambda i: (i, 0)),
            pl.BlockSpec((KPAD, HID), lambda i: (0, 0)),
            pl.BlockSpec((NC, RB, 8), lambda i: (0, i, 0)),
        ],
        out_specs=pl.BlockSpec((NC, RB, HH), lambda i: (0, i, 0)),
        out_shape=jax.ShapeDtypeStruct((NC, N_NODES, HH), jnp.float32),
    )(x, w1p, degp)


def _tcb_body(agg_ref, hwp_ref, deg_ref, b_ref, g_ref, be_ref, out_ref, ssum, ssq):
    p = pl.program_id(0)
    i = pl.program_id(1)
    dinv = _dinv_from(deg_ref[...])[None, :, None]
    t = (agg_ref[...] + hwp_ref[...]) * dinv + b_ref[...][:, None, :]

    @pl.when((p == 0) & (i == 0))
    def _():
        ssum[...] = jnp.zeros_like(ssum)
        ssq[...] = jnp.zeros_like(ssq)

    @pl.when(p == 0)
    def _():
        ssum[...] += t.sum(axis=1)

    @pl.when(p == 1)
    def _():
        mu = (ssum[...] / N_NODES)[:, None, :]
        d = t - mu
        ssq[...] += (d * d).sum(axis=1)

    @pl.when(p == 2)
    def _():
        mu = (ssum[...] / N_NODES)[:, None, :]
        var = (ssq[...] / N_NODES)[:, None, :]
        z = g_ref[...][:, None, :] * (t - mu) / jnp.sqrt(var + BN_EPS) \
            + be_ref[...][:, None, :]
        out_ref[...] = jnp.where(z > 0, z, ALPHA * z)


def _tcb(agg, hwp, degp, b, g, be):
    return pl.pallas_call(
        _tcb_body,
        grid=(3, NRB),
        in_specs=[
            pl.BlockSpec((NC, RB, HH), lambda p, i: (0, i, 0)),
            pl.BlockSpec((NC, RB, HH), lambda p, i: (0, i, 0)),
            pl.BlockSpec((NC, RB, 8), lambda p, i: (0, i, 0)),
            pl.BlockSpec((NC, HH), lambda p, i: (0, 0)),
            pl.BlockSpec((NC, HH), lambda p, i: (0, 0)),
            pl.BlockSpec((NC, HH), lambda p, i: (0, 0)),
        ],
        out_specs=pl.BlockSpec((NC, RB, HH), lambda p, i: (0, i, 0)),
        out_shape=jax.ShapeDtypeStruct((NC, N_NODES, HH), jnp.float32),
        scratch_shapes=[
            pltpu.VMEM((NC, HH), jnp.float32),
            pltpu.VMEM((NC, HH), jnp.float32),
        ],
    )(agg, hwp, degp, b, g, be)


def _tcbw_body(agg_ref, hwp_ref, deg_ref, b_ref, g_ref, be_ref, w_ref,
               out_ref, ssum, ssq):
    p = pl.program_id(0)
    i = pl.program_id(1)
    dinv1 = _dinv_from(deg_ref[...])
    dinv = dinv1[None, :, None]
    t = (agg_ref[...] + hwp_ref[...]) * dinv + b_ref[...][:, None, :]

    @pl.when((p == 0) & (i == 0))
    def _():
        ssum[...] = jnp.zeros_like(ssum)
        ssq[...] = jnp.zeros_like(ssq)

    @pl.when(p == 0)
    def _():
        ssum[...] += t.sum(axis=1)

    @pl.when(p == 1)
    def _():
        mu = (ssum[...] / N_NODES)[:, None, :]
        d = t - mu
        ssq[...] += (d * d).sum(axis=1)

    @pl.when(p == 2)
    def _():
        mu = (ssum[...] / N_NODES)[:, None, :]
        var = (ssq[...] / N_NODES)[:, None, :]
        z = g_ref[...][:, None, :] * (t - mu) / jnp.sqrt(var + BN_EPS) \
            + be_ref[...][:, None, :]
        h = jnp.where(z > 0, z, ALPHA * z)
        hh = jnp.concatenate([h[0], h[1]], axis=1)
        t2 = jnp.dot(hh, w_ref[...], preferred_element_type=jnp.float32)
        t2 = t2 * dinv1[:, None]
        out_ref[0] = t2[:, :HH]
        out_ref[1] = t2[:, HH:]


def _tcbw(agg, hwp, degp, b, g, be, w):
    return pl.pallas_call(
        _tcbw_body,
        grid=(3, NRB),
        in_specs=[
            pl.BlockSpec((NC, RB, HH), lambda p, i: (0, i, 0)),
            pl.BlockSpec((NC, RB, HH), lambda p, i: (0, i, 0)),
            pl.BlockSpec((NC, RB, 8), lambda p, i: (0, i, 0)),
            pl.BlockSpec((NC, HH), lambda p, i: (0, 0)),
            pl.BlockSpec((NC, HH), lambda p, i: (0, 0)),
            pl.BlockSpec((NC, HH), lambda p, i: (0, 0)),
            pl.BlockSpec((HID, HID), lambda p, i: (0, 0)),
        ],
        out_specs=pl.BlockSpec((NC, RB, HH), lambda p, i: (0, i, 0)),
        out_shape=jax.ShapeDtypeStruct((NC, N_NODES, HH), jnp.float32),
        scratch_shapes=[
            pltpu.VMEM((NC, HH), jnp.float32),
            pltpu.VMEM((NC, HH), jnp.float32),
        ],
    )(agg, hwp, degp, b, g, be, w)


def _mlp_body(gp_ref, w1_ref, b1_ref, g1_ref, be1_ref,
              w2_ref, b2_ref, g2_ref, be2_ref, w3_ref, b3_ref, out_ref):
    gp = gp_ref[...]
    g = jnp.concatenate([gp[0], gp[1]], axis=1)

    def bn(z, ga, be):
        mu = jnp.mean(z, axis=0)
        d = z - mu
        var = jnp.mean(d * d, axis=0)
        return ga * (z - mu) / jnp.sqrt(var + BN_EPS) + be

    z = jnp.dot(g, w1_ref[...], preferred_element_type=jnp.float32) + b1_ref[...]
    z = jnp.maximum(bn(z, g1_ref[...], be1_ref[...]), 0.0)
    z = jnp.dot(z, w2_ref[...], preferred_element_type=jnp.float32) + b2_ref[...]
    z = jnp.maximum(bn(z, g2_ref[...], be2_ref[...]), 0.0)
    out_ref[...] = jnp.dot(z, w3_ref[...], preferred_element_type=jnp.float32) + b3_ref[...]


def _mlp(gparts, m):
    args = (gparts, m['W1'], m['b1'].reshape(1, -1), m['g1'].reshape(1, -1),
            m['be1'].reshape(1, -1), m['W2'], m['b2'].reshape(1, -1),
            m['g2'].reshape(1, -1), m['be2'].reshape(1, -1), m['W3'],
            m['b3'].reshape(1, -1))
    return pl.pallas_call(
        _mlp_body,
        out_shape=jax.ShapeDtypeStruct((NG, 2), jnp.float32),
    )(*args)


def kernel(x, edge_index, batch, params):
    ei = edge_index.astype(jnp.int32)
    src, dst = ei[0], ei[1]
    pad = E_PAD - N_EDGES
    srcp = jnp.concatenate([src, jnp.zeros((pad,), jnp.int32)])
    dstp = jnp.concatenate([dst, jnp.full((pad,), DUMMY, jnp.int32)])
    eidx = jnp.stack([srcp.reshape(NS, NCHT, ECH),
                      dstp.reshape(NS, NCHT, ECH)], axis=2)
    bat = batch.astype(jnp.int32)
    batT = bat[:TOFF].reshape(NS, NPJ, PCH)
    batTail = bat[TOFF:].reshape(1, TAIL)
    zeros = jnp.zeros((RPS, HH), jnp.float32)
    ones128 = jnp.ones((ECH, HH), jnp.float32)

    w1p = jnp.concatenate([params['gcn'][0]['W'],
                           jnp.zeros((KPAD - IN_DIM, HID), jnp.float32)], axis=0)

    degp = _sc_deg(eidx, ones128, zeros)
    degp = degp[:, :, :8]
    hwp = _tca1(x.astype(jnp.int32), w1p, degp)
    h = None
    for li, lp in enumerate(params['gcn']):
        agg = _sc_agg(hwp, eidx, zeros)
        bb = lp['b'].reshape(NC, HH)
        gg = lp['gamma'].reshape(NC, HH)
        be = lp['beta'].reshape(NC, HH)
        if li < len(params['gcn']) - 1:
            hwp = _tcbw(agg, hwp, degp, bb, gg, be, params['gcn'][li + 1]['W'])
        else:
            h = _tcb(agg, hwp, degp, bb, gg, be)
    g = _sc_pool(h, batT, batTail, zeros)
    return _mlp(g, params['mlp'])
